# baseline (speedup 1.0000x reference)
import functools

import jax
import jax.numpy as jnp
from jax import lax
from jax.experimental import pallas as pl
from jax.experimental.pallas import tpu as pltpu
from jax.experimental.pallas import tpu_sc as plsc

N = 10000
E = 320000
D_NODE_DIM = 128
HID = 128
NC = 2
NS = 16
NW = NC * NS
EPW = E // NW
CHUNK = 80
NCHUNK = EPW // CHUNK
GCH = 400
GNC = EPW // GCH
SCH = 200
SNC = EPW // SCH
ROWS_A = 632
ROWS_B = N - (NS - 1) * ROWS_A

_sc_mesh = plsc.VectorSubcoreMesh(core_axis_name="c", subcore_axis_name="s")


def _mm_bias_body(a_ref, w_ref, b_ref, o_ref):
    o_ref[...] = (
        jnp.dot(a_ref[...], w_ref[...], preferred_element_type=jnp.float32)
        + b_ref[...]
    )


def _mm_body(a_ref, w_ref, o_ref):
    o_ref[...] = jnp.dot(a_ref[...], w_ref[...], preferred_element_type=jnp.float32)


def _tc_matmul(a, w, bias=None, block=4000):
    m, k = a.shape
    n = w.shape[1]
    grid = (m // block,)
    in_specs = [
        pl.BlockSpec((block, k), lambda i: (i, 0)),
        pl.BlockSpec((k, n), lambda i: (0, 0)),
    ]
    args = [a, w]
    body = _mm_body
    if bias is not None:
        in_specs.append(pl.BlockSpec((1, n), lambda i: (0, 0)))
        args.append(bias)
        body = _mm_bias_body
    return pl.pallas_call(
        body,
        grid=grid,
        in_specs=in_specs,
        out_specs=pl.BlockSpec((block, n), lambda i: (i, 0)),
        out_shape=jax.ShapeDtypeStruct((m, n), jnp.float32),
    )(*args)


def _mm_relu_add_body(q_ref, d_ref, w_ref, o_ref):
    h = jnp.maximum(q_ref[...] + d_ref[...], 0.0)
    o_ref[...] = jnp.dot(h, w_ref[...], preferred_element_type=jnp.float32)


def _mm_relu_add(q, d, w, block=4000):
    m = q.shape[0]
    n = w.shape[1]
    grid = (m // block,)
    return pl.pallas_call(
        _mm_relu_add_body,
        grid=grid,
        in_specs=[
            pl.BlockSpec((block, HID), lambda i: (i, 0)),
            pl.BlockSpec((block, HID), lambda i: (i, 0)),
            pl.BlockSpec((HID, n), lambda i: (0, 0)),
        ],
        out_specs=pl.BlockSpec((block, n), lambda i: (i, 0)),
        out_shape=jax.ShapeDtypeStruct((m, n), jnp.float32),
    )(q, d, w)


def _combine_body(ab_ref, p_ref, bh_ref, t_ref):
    t_ref[...] = ab_ref[0] + ab_ref[1] + p_ref[...] + bh_ref[...]


def _combine(ab, p, bh, block=2000):
    grid = (N // block,)
    return pl.pallas_call(
        _combine_body,
        grid=grid,
        in_specs=[
            pl.BlockSpec((NC, block, HID), lambda i: (0, i, 0)),
            pl.BlockSpec((block, HID), lambda i: (i, 0)),
            pl.BlockSpec((1, HID), lambda i: (0, 0)),
        ],
        out_specs=pl.BlockSpec((block, HID), lambda i: (i, 0)),
        out_shape=jax.ShapeDtypeStruct((N, HID), jnp.float32),
    )(ab, p, bh)


def _final_body(x_ref, ab_ref, w1_ref, w2_ref, b_ref, o_ref):
    f = ab_ref[0] + ab_ref[1]
    cond = jnp.sum(f, axis=1, keepdims=True) == 0.0
    mp = jnp.where(cond, x_ref[...], f)
    o_ref[...] = jax.nn.relu(
        jnp.dot(x_ref[...], w1_ref[...], preferred_element_type=jnp.float32)
        + jnp.dot(mp, w2_ref[...], preferred_element_type=jnp.float32)
        + b_ref[...]
    )


def _final(x, ab, w1t, w2t, bo, block=2000):
    grid = (N // block,)
    return pl.pallas_call(
        _final_body,
        grid=grid,
        in_specs=[
            pl.BlockSpec((block, D_NODE_DIM), lambda i: (i, 0)),
            pl.BlockSpec((NC, block, HID), lambda i: (0, i, 0)),
            pl.BlockSpec((D_NODE_DIM, HID), lambda i: (0, 0)),
            pl.BlockSpec((HID, HID), lambda i: (0, 0)),
            pl.BlockSpec((1, HID), lambda i: (0, 0)),
        ],
        out_specs=pl.BlockSpec((block, HID), lambda i: (i, 0)),
        out_shape=jax.ShapeDtypeStruct((N, HID), jnp.float32),
    )(x, ab, w1t, w2t, bo)


def _zero_acc(zbuf, acc_sh, s, row_off):
    def zrow(i, carry):
        for j in range(HID // 16):
            zbuf[i, pl.ds(j * 16, 16)] = jnp.zeros((16,), jnp.float32)
        return carry

    lax.fori_loop(0, CHUNK, zrow, 0)

    @pl.when(s < NS - 1)
    def _():
        def zfill(i, carry):
            off = pl.multiple_of(s * ROWS_A + i * CHUNK, 8)
            pltpu.sync_copy(zbuf, acc_sh.at[pl.ds(off, CHUNK)])
            return carry
        lax.fori_loop(0, ROWS_A // CHUNK, zfill, 0)
        pltpu.sync_copy(zbuf.at[pl.ds(0, ROWS_A % CHUNK)],
                        acc_sh.at[pl.ds(pl.multiple_of(
                            s * ROWS_A + (ROWS_A // CHUNK) * CHUNK, 8),
                            ROWS_A % CHUNK)])

    @pl.when(s == NS - 1)
    def _():
        base_b = (NS - 1) * ROWS_A

        def zfill(i, carry):
            off = pl.multiple_of(base_b + i * CHUNK, 8)
            pltpu.sync_copy(zbuf, acc_sh.at[pl.ds(off, CHUNK)])
            return carry
        lax.fori_loop(0, ROWS_B // CHUNK, zfill, 0)
        pltpu.sync_copy(zbuf.at[pl.ds(0, ROWS_B % CHUNK)],
                        acc_sh.at[pl.ds(base_b + (ROWS_B // CHUNK) * CHUNK,
                                        ROWS_B % CHUNK)])


def _readout_acc(acc_sh, out_hbm, c, s, row_off):
    @pl.when(s < NS - 1)
    def _():
        pltpu.sync_copy(acc_sh.at[pl.ds(row_off, ROWS_A)],
                        out_hbm.at[c, pl.ds(row_off, ROWS_A)])

    @pl.when(s == NS - 1)
    def _():
        pltpu.sync_copy(acc_sh.at[pl.ds((NS - 1) * ROWS_A, ROWS_B)],
                        out_hbm.at[c, pl.ds((NS - 1) * ROWS_A, ROWS_B)])


@functools.partial(
    pl.kernel,
    out_type=jax.ShapeDtypeStruct((NC, N, HID), jnp.float32),
    mesh=_sc_mesh,
    scratch_types=[
        pltpu.VMEM((CHUNK,), jnp.int32),
        pltpu.VMEM((CHUNK,), jnp.int32),
        pltpu.VMEM((CHUNK, HID), jnp.float32),
        pltpu.VMEM((CHUNK, HID), jnp.float32),
        pltpu.VMEM_SHARED((N, HID), jnp.float32),
        pltpu.SemaphoreType.DMA,
        pltpu.SemaphoreType.DMA,
        pltpu.SemaphoreType.DMA,
        pltpu.SemaphoreType.DMA,
    ],
)
def _sc_scatter(rows_hbm, dst_hbm, out_hbm,
                idx0, idx1, rows0, rows1, acc_sh,
                sem_l0, sem_l1, sem_s0, sem_s1):
    c = lax.axis_index("c")
    s = lax.axis_index("s")
    wid = c * NS + s
    row_off = pl.multiple_of(s * ROWS_A, 8)
    idx_v = (idx0, idx1)
    rows_v = (rows0, rows1)
    sem_l = (sem_l0, sem_l1)
    sem_s = (sem_s0, sem_s1)

    _zero_acc(rows0, acc_sh, s, row_off)
    plsc.subcore_barrier()

    base0 = wid * EPW

    def cbase(k):
        return pl.multiple_of(base0 + k * CHUNK, 8)

    pltpu.async_copy(dst_hbm.at[pl.ds(cbase(0), CHUNK)], idx0, sem_l0)
    pltpu.async_copy(rows_hbm.at[pl.ds(cbase(0), CHUNK)], rows0, sem_l0)

    def pair(g, carry):
        for b in (0, 1):
            k = 2 * g + b
            nb = 1 - b

            @pl.when(k < NCHUNK)
            def _():
                pltpu.make_async_copy(
                    dst_hbm.at[pl.ds(cbase(k), CHUNK)], idx_v[b], sem_l[b]).wait()
                pltpu.make_async_copy(
                    rows_hbm.at[pl.ds(cbase(k), CHUNK)], rows_v[b], sem_l[b]).wait()

                @pl.when(k >= 1)
                def _():
                    pltpu.make_async_copy(
                        rows_hbm.at[pl.ds(cbase(0), CHUNK)], rows_v[nb],
                        sem_s[nb]).wait()

                @pl.when(k + 1 < NCHUNK)
                def _():
                    pltpu.async_copy(
                        dst_hbm.at[pl.ds(cbase(k + 1), CHUNK)], idx_v[nb], sem_l[nb])
                    pltpu.async_copy(
                        rows_hbm.at[pl.ds(cbase(k + 1), CHUNK)], rows_v[nb], sem_l[nb])

                pltpu.async_copy(
                    rows_v[b], acc_sh.at[idx_v[b]], sem_s[b], add=True)
        return carry

    lax.fori_loop(0, (NCHUNK + 1) // 2, pair, 0)
    pltpu.make_async_copy(
        rows_hbm.at[pl.ds(cbase(0), CHUNK)], rows_v[0], sem_s[0]).wait()

    plsc.subcore_barrier()
    _readout_acc(acc_sh, out_hbm, c, s, row_off)


@functools.partial(
    pl.kernel,
    out_type=jax.ShapeDtypeStruct((NC, N, HID), jnp.float32),
    mesh=_sc_mesh,
    scratch_types=[
        pltpu.VMEM((CHUNK,), jnp.int32),
        pltpu.VMEM((CHUNK,), jnp.int32),
        pltpu.VMEM((CHUNK, HID), jnp.float32),
        pltpu.VMEM((CHUNK, HID), jnp.float32),
        pltpu.VMEM((CHUNK, HID), jnp.float32),
        pltpu.VMEM((CHUNK, HID), jnp.float32),
        pltpu.VMEM_SHARED((N, HID), jnp.float32),
        pltpu.SemaphoreType.DMA,
        pltpu.SemaphoreType.DMA,
        pltpu.SemaphoreType.DMA,
        pltpu.SemaphoreType.DMA,
    ],
)
def _sc_scatter_relu(q_hbm, d_hbm, dst_hbm, out_hbm,
                     idx0, idx1, q0, q1, d0, d1, acc_sh,
                     sem_l0, sem_l1, sem_s0, sem_s1):
    c = lax.axis_index("c")
    s = lax.axis_index("s")
    wid = c * NS + s
    row_off = pl.multiple_of(s * ROWS_A, 8)
    idx_v = (idx0, idx1)
    q_v = (q0, q1)
    d_v = (d0, d1)
    sem_l = (sem_l0, sem_l1)
    sem_s = (sem_s0, sem_s1)

    _zero_acc(q0, acc_sh, s, row_off)
    plsc.subcore_barrier()

    base0 = wid * EPW

    def cbase(k):
        return pl.multiple_of(base0 + k * CHUNK, 8)

    pltpu.async_copy(dst_hbm.at[pl.ds(cbase(0), CHUNK)], idx0, sem_l0)
    pltpu.async_copy(q_hbm.at[pl.ds(cbase(0), CHUNK)], q0, sem_l0)
    pltpu.async_copy(d_hbm.at[pl.ds(cbase(0), CHUNK)], d0, sem_l0)

    def pair(g, carry):
        for b in (0, 1):
            k = 2 * g + b
            nb = 1 - b

            @pl.when(k < NCHUNK)
            def _():
                pltpu.make_async_copy(
                    dst_hbm.at[pl.ds(cbase(k), CHUNK)], idx_v[b], sem_l[b]).wait()
                pltpu.make_async_copy(
                    q_hbm.at[pl.ds(cbase(k), CHUNK)], q_v[b], sem_l[b]).wait()
                pltpu.make_async_copy(
                    d_hbm.at[pl.ds(cbase(k), CHUNK)], d_v[b], sem_l[b]).wait()

                @pl.when(k >= 1)
                def _():
                    pltpu.make_async_copy(
                        q_hbm.at[pl.ds(cbase(0), CHUNK)], q_v[nb],
                        sem_s[nb]).wait()

                @pl.when(k + 1 < NCHUNK)
                def _():
                    pltpu.async_copy(
                        dst_hbm.at[pl.ds(cbase(k + 1), CHUNK)], idx_v[nb], sem_l[nb])
                    pltpu.async_copy(
                        q_hbm.at[pl.ds(cbase(k + 1), CHUNK)], q_v[nb], sem_l[nb])
                    pltpu.async_copy(
                        d_hbm.at[pl.ds(cbase(k + 1), CHUNK)], d_v[nb], sem_l[nb])

                def row2(i, carry2):
                    for r in range(2):
                        for j in range(HID // 16):
                            sl = pl.ds(j * 16, 16)
                            q_v[b][i * 2 + r, sl] = jnp.maximum(
                                q_v[b][i * 2 + r, sl] + d_v[b][i * 2 + r, sl], 0.0)
                    return carry2

                lax.fori_loop(0, CHUNK // 2, row2, 0)

                pltpu.async_copy(
                    q_v[b], acc_sh.at[idx_v[b]], sem_s[b], add=True)
        return carry

    lax.fori_loop(0, (NCHUNK + 1) // 2, pair, 0)
    pltpu.make_async_copy(
        q_hbm.at[pl.ds(cbase(0), CHUNK)], q_v[0], sem_s[0]).wait()

    plsc.subcore_barrier()
    _readout_acc(acc_sh, out_hbm, c, s, row_off)


@functools.partial(
    pl.kernel,
    out_type=jax.ShapeDtypeStruct((E, HID), jnp.float32),
    mesh=_sc_mesh,
    scratch_types=(
        [pltpu.VMEM((GCH,), jnp.int32)] * 2
        + [pltpu.VMEM((GCH, HID), jnp.float32)] * 2
        + [pltpu.SemaphoreType.DMA] * 6
    ),
)
def _sc_gather(tab_hbm, src_hbm, out_hbm,
               i0, i1, t0, t1,
               si0, si1, sg0, sg1, so0, so1):
    c = lax.axis_index("c")
    s = lax.axis_index("s")
    base0 = (c * NS + s) * EPW
    idx_v = (i0, i1)
    t_v = (t0, t1)
    sem_i = (si0, si1)
    sem_g = (sg0, sg1)
    sem_o = (so0, so1)

    def cbase(k):
        return pl.multiple_of(base0 + k * GCH, 8)

    pltpu.async_copy(src_hbm.at[pl.ds(cbase(0), GCH)], i0, si0)

    def pair(g, carry):
        for b in (0, 1):
            k = 2 * g + b
            nb = 1 - b

            @pl.when(k < GNC)
            def _():
                pltpu.make_async_copy(
                    src_hbm.at[pl.ds(cbase(k), GCH)], idx_v[b], sem_i[b]).wait()

                @pl.when(k >= 2)
                def _():
                    pltpu.make_async_copy(
                        t_v[b], out_hbm.at[pl.ds(cbase(k - 2), GCH)],
                        sem_o[b]).wait()

                cps = [
                    pltpu.async_copy(
                        tab_hbm.at[idx_v[b].at[pl.ds(j * 80, 80)]],
                        t_v[b].at[pl.ds(j * 80, 80)], sem_g[b])
                    for j in range(GCH // 80)
                ]

                @pl.when(k + 1 < GNC)
                def _():
                    pltpu.async_copy(
                        src_hbm.at[pl.ds(cbase(k + 1), GCH)], idx_v[nb], sem_i[nb])

                @pl.when(k >= 1)
                def _():
                    pltpu.async_copy(
                        t_v[nb], out_hbm.at[pl.ds(cbase(k - 1), GCH)], sem_o[nb])

                for cp in cps:
                    cp.wait()

            @pl.when(k == GNC)
            def _():
                pltpu.async_copy(
                    t_v[nb], out_hbm.at[pl.ds(cbase(GNC - 1), GCH)], sem_o[nb])
        return carry

    lax.fori_loop(0, (GNC + 2) // 2, pair, 0)
    pltpu.make_async_copy(
        t_v[1], out_hbm.at[pl.ds(cbase(GNC - 2), GCH)], sem_o[1]).wait()
    pltpu.make_async_copy(
        t_v[0], out_hbm.at[pl.ds(cbase(GNC - 1), GCH)], sem_o[0]).wait()


_SUBG = ((0, 80), (80, 80), (160, 40))


@functools.partial(
    pl.kernel,
    out_type=jax.ShapeDtypeStruct((E, HID), jnp.float32),
    mesh=_sc_mesh,
    scratch_types=(
        [pltpu.VMEM((SCH,), jnp.int32)] * 4
        + [pltpu.VMEM((SCH, HID), jnp.float32)] * 4
        + [pltpu.SemaphoreType.DMA] * 6
    ),
)
def _sc_gather_sub(tab_hbm, g_hbm, src_hbm, rev_hbm, out_hbm,
                   a0, a1, r0, r1, t0, t1, g0, g1,
                   si0, si1, sg0, sg1, so0, so1):
    c = lax.axis_index("c")
    s = lax.axis_index("s")
    base0 = (c * NS + s) * EPW
    sidx_v = (a0, a1)
    ridx_v = (r0, r1)
    t_v = (t0, t1)
    g_v = (g0, g1)
    sem_i = (si0, si1)
    sem_g = (sg0, sg1)
    sem_o = (so0, so1)

    def cbase(k):
        return pl.multiple_of(base0 + k * SCH, 8)

    pltpu.async_copy(src_hbm.at[pl.ds(cbase(0), SCH)], a0, si0)
    pltpu.async_copy(rev_hbm.at[pl.ds(cbase(0), SCH)], r0, si0)

    def pair(g, carry):
        for b in (0, 1):
            k = 2 * g + b
            nb = 1 - b

            @pl.when(k < SNC)
            def _():
                pltpu.make_async_copy(
                    src_hbm.at[pl.ds(cbase(k), SCH)], sidx_v[b], sem_i[b]).wait()
                pltpu.make_async_copy(
                    rev_hbm.at[pl.ds(cbase(k), SCH)], ridx_v[b], sem_i[b]).wait()

                @pl.when(k >= 2)
                def _():
                    pltpu.make_async_copy(
                        t_v[b], out_hbm.at[pl.ds(cbase(k - 2), SCH)],
                        sem_o[b]).wait()

                cps = []
                for off, ln in _SUBG:
                    cps.append(pltpu.async_copy(
                        tab_hbm.at[sidx_v[b].at[pl.ds(off, ln)]],
                        t_v[b].at[pl.ds(off, ln)], sem_g[b]))
                    cps.append(pltpu.async_copy(
                        g_hbm.at[ridx_v[b].at[pl.ds(off, ln)]],
                        g_v[b].at[pl.ds(off, ln)], sem_g[b]))

                @pl.when(k + 1 < SNC)
                def _():
                    pltpu.async_copy(
                        src_hbm.at[pl.ds(cbase(k + 1), SCH)], sidx_v[nb], sem_i[nb])
                    pltpu.async_copy(
                        rev_hbm.at[pl.ds(cbase(k + 1), SCH)], ridx_v[nb], sem_i[nb])

                @pl.when(k >= 1)
                def _():
                    def row2(i, carry2):
                        for r in range(2):
                            for j in range(HID // 16):
                                sl = pl.ds(j * 16, 16)
                                t_v[nb][i * 2 + r, sl] = (
                                    t_v[nb][i * 2 + r, sl]
                                    - g_v[nb][i * 2 + r, sl])
                        return carry2

                    lax.fori_loop(0, SCH // 2, row2, 0)
                    pltpu.async_copy(
                        t_v[nb], out_hbm.at[pl.ds(cbase(k - 1), SCH)], sem_o[nb])

                for cp in cps:
                    cp.wait()

            @pl.when(k == SNC)
            def _():
                def row2(i, carry2):
                    for r in range(2):
                        for j in range(HID // 16):
                            sl = pl.ds(j * 16, 16)
                            t_v[nb][i * 2 + r, sl] = (
                                t_v[nb][i * 2 + r, sl] - g_v[nb][i * 2 + r, sl])
                    return carry2

                lax.fori_loop(0, SCH // 2, row2, 0)
                pltpu.async_copy(
                    t_v[nb], out_hbm.at[pl.ds(cbase(SNC - 1), SCH)], sem_o[nb])
        return carry

    lax.fori_loop(0, (SNC + 2) // 2, pair, 0)
    pltpu.make_async_copy(
        t_v[1], out_hbm.at[pl.ds(cbase(SNC - 2), SCH)], sem_o[1]).wait()
    pltpu.make_async_copy(
        t_v[0], out_hbm.at[pl.ds(cbase(SNC - 1), SCH)], sem_o[0]).wait()


def kernel(x, edge_index, edge_attr, rev_edge_index, W_i, b_i, W_h, b_h, W_o, b_o):
    src = edge_index[0]
    dst = edge_index[1]
    wxt = jnp.transpose(W_i[:, :D_NODE_DIM])
    wet = jnp.transpose(W_i[:, D_NODE_DIM:])
    wht = jnp.transpose(W_h)
    wo1t = jnp.transpose(W_o[:, :D_NODE_DIM])
    wo2t = jnp.transpose(W_o[:, D_NODE_DIM:])
    bi2 = b_i.reshape(1, HID)
    bh2 = b_h.reshape(1, HID)
    bo2 = b_o.reshape(1, HID)

    p = _tc_matmul(x, wxt, block=2000)
    q = _tc_matmul(edge_attr, wet, bias=bi2)
    d = _sc_gather(p, src)
    g = _mm_relu_add(q, d, wht)
    for t in range(2):
        ab = _sc_scatter(g, dst)
        tt = _combine(ab, p, bh2)
        d = _sc_gather_sub(tt, g, src, rev_edge_index)
        if t == 0:
            g = _mm_relu_add(q, d, wht)
    ab = _sc_scatter_relu(q, d, dst)
    return _final(x, ab, wo1t, wo2t, bo2)


# speedup vs baseline: 1.0156x; 1.0156x over previous
import functools

import jax
import jax.numpy as jnp
from jax import lax
from jax.experimental import pallas as pl
from jax.experimental.pallas import tpu as pltpu
from jax.experimental.pallas import tpu_sc as plsc

N = 10000
E = 320000
D_NODE_DIM = 128
HID = 128
NC = 2
NS = 16
NW = NC * NS
EPW = E // NW
CHUNK = 80
NCHUNK = EPW // CHUNK
GCH = 400
GNC = EPW // GCH
SCH = 200
SNC = EPW // SCH
ROWS_A = 632
ROWS_B = N - (NS - 1) * ROWS_A

_sc_mesh = plsc.VectorSubcoreMesh(core_axis_name="c", subcore_axis_name="s")


def _mm_bias_body(a_ref, w_ref, b_ref, o_ref):
    o_ref[...] = (
        jnp.dot(a_ref[...], w_ref[...], preferred_element_type=jnp.float32)
        + b_ref[...]
    )


def _mm_body(a_ref, w_ref, o_ref):
    o_ref[...] = jnp.dot(a_ref[...], w_ref[...], preferred_element_type=jnp.float32)


def _tc_matmul(a, w, bias=None, block=8000):
    m, k = a.shape
    n = w.shape[1]
    grid = (m // block,)
    in_specs = [
        pl.BlockSpec((block, k), lambda i: (i, 0)),
        pl.BlockSpec((k, n), lambda i: (0, 0)),
    ]
    args = [a, w]
    body = _mm_body
    if bias is not None:
        in_specs.append(pl.BlockSpec((1, n), lambda i: (0, 0)))
        args.append(bias)
        body = _mm_bias_body
    return pl.pallas_call(
        body,
        grid=grid,
        in_specs=in_specs,
        out_specs=pl.BlockSpec((block, n), lambda i: (i, 0)),
        out_shape=jax.ShapeDtypeStruct((m, n), jnp.float32),
    )(*args)


def _mm_relu_add_body(q_ref, d_ref, w_ref, o_ref):
    h = jnp.maximum(q_ref[...] + d_ref[...], 0.0)
    o_ref[...] = jnp.dot(h, w_ref[...], preferred_element_type=jnp.float32)


def _mm_relu_add(q, d, w, block=8000):
    m = q.shape[0]
    n = w.shape[1]
    grid = (m // block,)
    return pl.pallas_call(
        _mm_relu_add_body,
        grid=grid,
        in_specs=[
            pl.BlockSpec((block, HID), lambda i: (i, 0)),
            pl.BlockSpec((block, HID), lambda i: (i, 0)),
            pl.BlockSpec((HID, n), lambda i: (0, 0)),
        ],
        out_specs=pl.BlockSpec((block, n), lambda i: (i, 0)),
        out_shape=jax.ShapeDtypeStruct((m, n), jnp.float32),
    )(q, d, w)


def _combine_body(ab_ref, p_ref, bh_ref, t_ref):
    t_ref[...] = ab_ref[0] + ab_ref[1] + p_ref[...] + bh_ref[...]


def _combine(ab, p, bh, block=2000):
    grid = (N // block,)
    return pl.pallas_call(
        _combine_body,
        grid=grid,
        in_specs=[
            pl.BlockSpec((NC, block, HID), lambda i: (0, i, 0)),
            pl.BlockSpec((block, HID), lambda i: (i, 0)),
            pl.BlockSpec((1, HID), lambda i: (0, 0)),
        ],
        out_specs=pl.BlockSpec((block, HID), lambda i: (i, 0)),
        out_shape=jax.ShapeDtypeStruct((N, HID), jnp.float32),
    )(ab, p, bh)


def _final_body(x_ref, ab_ref, w1_ref, w2_ref, b_ref, o_ref):
    f = ab_ref[0] + ab_ref[1]
    cond = jnp.sum(f, axis=1, keepdims=True) == 0.0
    mp = jnp.where(cond, x_ref[...], f)
    o_ref[...] = jax.nn.relu(
        jnp.dot(x_ref[...], w1_ref[...], preferred_element_type=jnp.float32)
        + jnp.dot(mp, w2_ref[...], preferred_element_type=jnp.float32)
        + b_ref[...]
    )


def _final(x, ab, w1t, w2t, bo, block=2000):
    grid = (N // block,)
    return pl.pallas_call(
        _final_body,
        grid=grid,
        in_specs=[
            pl.BlockSpec((block, D_NODE_DIM), lambda i: (i, 0)),
            pl.BlockSpec((NC, block, HID), lambda i: (0, i, 0)),
            pl.BlockSpec((D_NODE_DIM, HID), lambda i: (0, 0)),
            pl.BlockSpec((HID, HID), lambda i: (0, 0)),
            pl.BlockSpec((1, HID), lambda i: (0, 0)),
        ],
        out_specs=pl.BlockSpec((block, HID), lambda i: (i, 0)),
        out_shape=jax.ShapeDtypeStruct((N, HID), jnp.float32),
    )(x, ab, w1t, w2t, bo)


def _zero_acc(zbuf, acc_sh, s, row_off):
    def zrow(i, carry):
        for j in range(HID // 16):
            zbuf[i, pl.ds(j * 16, 16)] = jnp.zeros((16,), jnp.float32)
        return carry

    lax.fori_loop(0, CHUNK, zrow, 0)

    @pl.when(s < NS - 1)
    def _():
        def zfill(i, carry):
            off = pl.multiple_of(s * ROWS_A + i * CHUNK, 8)
            pltpu.sync_copy(zbuf, acc_sh.at[pl.ds(off, CHUNK)])
            return carry
        lax.fori_loop(0, ROWS_A // CHUNK, zfill, 0)
        pltpu.sync_copy(zbuf.at[pl.ds(0, ROWS_A % CHUNK)],
                        acc_sh.at[pl.ds(pl.multiple_of(
                            s * ROWS_A + (ROWS_A // CHUNK) * CHUNK, 8),
                            ROWS_A % CHUNK)])

    @pl.when(s == NS - 1)
    def _():
        base_b = (NS - 1) * ROWS_A

        def zfill(i, carry):
            off = pl.multiple_of(base_b + i * CHUNK, 8)
            pltpu.sync_copy(zbuf, acc_sh.at[pl.ds(off, CHUNK)])
            return carry
        lax.fori_loop(0, ROWS_B // CHUNK, zfill, 0)
        pltpu.sync_copy(zbuf.at[pl.ds(0, ROWS_B % CHUNK)],
                        acc_sh.at[pl.ds(base_b + (ROWS_B // CHUNK) * CHUNK,
                                        ROWS_B % CHUNK)])


def _readout_acc(acc_sh, out_hbm, c, s, row_off):
    @pl.when(s < NS - 1)
    def _():
        pltpu.sync_copy(acc_sh.at[pl.ds(row_off, ROWS_A)],
                        out_hbm.at[c, pl.ds(row_off, ROWS_A)])

    @pl.when(s == NS - 1)
    def _():
        pltpu.sync_copy(acc_sh.at[pl.ds((NS - 1) * ROWS_A, ROWS_B)],
                        out_hbm.at[c, pl.ds((NS - 1) * ROWS_A, ROWS_B)])


@functools.partial(
    pl.kernel,
    out_type=jax.ShapeDtypeStruct((NC, N, HID), jnp.float32),
    mesh=_sc_mesh,
    scratch_types=[
        pltpu.VMEM((CHUNK,), jnp.int32),
        pltpu.VMEM((CHUNK,), jnp.int32),
        pltpu.VMEM((CHUNK, HID), jnp.float32),
        pltpu.VMEM((CHUNK, HID), jnp.float32),
        pltpu.VMEM_SHARED((N, HID), jnp.float32),
        pltpu.SemaphoreType.DMA,
        pltpu.SemaphoreType.DMA,
        pltpu.SemaphoreType.DMA,
        pltpu.SemaphoreType.DMA,
    ],
)
def _sc_scatter(rows_hbm, dst_hbm, out_hbm,
                idx0, idx1, rows0, rows1, acc_sh,
                sem_l0, sem_l1, sem_s0, sem_s1):
    c = lax.axis_index("c")
    s = lax.axis_index("s")
    wid = c * NS + s
    row_off = pl.multiple_of(s * ROWS_A, 8)
    idx_v = (idx0, idx1)
    rows_v = (rows0, rows1)
    sem_l = (sem_l0, sem_l1)
    sem_s = (sem_s0, sem_s1)

    _zero_acc(rows0, acc_sh, s, row_off)
    plsc.subcore_barrier()

    base0 = wid * EPW

    def cbase(k):
        return pl.multiple_of(base0 + k * CHUNK, 8)

    pltpu.async_copy(dst_hbm.at[pl.ds(cbase(0), CHUNK)], idx0, sem_l0)
    pltpu.async_copy(rows_hbm.at[pl.ds(cbase(0), CHUNK)], rows0, sem_l0)

    def pair(g, carry):
        for b in (0, 1):
            k = 2 * g + b
            nb = 1 - b

            @pl.when(k < NCHUNK)
            def _():
                pltpu.make_async_copy(
                    dst_hbm.at[pl.ds(cbase(k), CHUNK)], idx_v[b], sem_l[b]).wait()
                pltpu.make_async_copy(
                    rows_hbm.at[pl.ds(cbase(k), CHUNK)], rows_v[b], sem_l[b]).wait()

                @pl.when(k >= 1)
                def _():
                    pltpu.make_async_copy(
                        rows_hbm.at[pl.ds(cbase(0), CHUNK)], rows_v[nb],
                        sem_s[nb]).wait()

                @pl.when(k + 1 < NCHUNK)
                def _():
                    pltpu.async_copy(
                        dst_hbm.at[pl.ds(cbase(k + 1), CHUNK)], idx_v[nb], sem_l[nb])
                    pltpu.async_copy(
                        rows_hbm.at[pl.ds(cbase(k + 1), CHUNK)], rows_v[nb], sem_l[nb])

                pltpu.async_copy(
                    rows_v[b], acc_sh.at[idx_v[b]], sem_s[b], add=True)
        return carry

    lax.fori_loop(0, (NCHUNK + 1) // 2, pair, 0)
    pltpu.make_async_copy(
        rows_hbm.at[pl.ds(cbase(0), CHUNK)], rows_v[0], sem_s[0]).wait()

    plsc.subcore_barrier()
    _readout_acc(acc_sh, out_hbm, c, s, row_off)


@functools.partial(
    pl.kernel,
    out_type=jax.ShapeDtypeStruct((NC, N, HID), jnp.float32),
    mesh=_sc_mesh,
    scratch_types=[
        pltpu.VMEM((CHUNK,), jnp.int32),
        pltpu.VMEM((CHUNK,), jnp.int32),
        pltpu.VMEM((CHUNK, HID), jnp.float32),
        pltpu.VMEM((CHUNK, HID), jnp.float32),
        pltpu.VMEM((CHUNK, HID), jnp.float32),
        pltpu.VMEM((CHUNK, HID), jnp.float32),
        pltpu.VMEM_SHARED((N, HID), jnp.float32),
        pltpu.SemaphoreType.DMA,
        pltpu.SemaphoreType.DMA,
        pltpu.SemaphoreType.DMA,
        pltpu.SemaphoreType.DMA,
    ],
)
def _sc_scatter_relu(q_hbm, d_hbm, dst_hbm, out_hbm,
                     idx0, idx1, q0, q1, d0, d1, acc_sh,
                     sem_l0, sem_l1, sem_s0, sem_s1):
    c = lax.axis_index("c")
    s = lax.axis_index("s")
    wid = c * NS + s
    row_off = pl.multiple_of(s * ROWS_A, 8)
    idx_v = (idx0, idx1)
    q_v = (q0, q1)
    d_v = (d0, d1)
    sem_l = (sem_l0, sem_l1)
    sem_s = (sem_s0, sem_s1)

    _zero_acc(q0, acc_sh, s, row_off)
    plsc.subcore_barrier()

    base0 = wid * EPW

    def cbase(k):
        return pl.multiple_of(base0 + k * CHUNK, 8)

    pltpu.async_copy(dst_hbm.at[pl.ds(cbase(0), CHUNK)], idx0, sem_l0)
    pltpu.async_copy(q_hbm.at[pl.ds(cbase(0), CHUNK)], q0, sem_l0)
    pltpu.async_copy(d_hbm.at[pl.ds(cbase(0), CHUNK)], d0, sem_l0)

    def pair(g, carry):
        for b in (0, 1):
            k = 2 * g + b
            nb = 1 - b

            @pl.when(k < NCHUNK)
            def _():
                pltpu.make_async_copy(
                    dst_hbm.at[pl.ds(cbase(k), CHUNK)], idx_v[b], sem_l[b]).wait()
                pltpu.make_async_copy(
                    q_hbm.at[pl.ds(cbase(k), CHUNK)], q_v[b], sem_l[b]).wait()
                pltpu.make_async_copy(
                    d_hbm.at[pl.ds(cbase(k), CHUNK)], d_v[b], sem_l[b]).wait()

                @pl.when(k >= 1)
                def _():
                    pltpu.make_async_copy(
                        q_hbm.at[pl.ds(cbase(0), CHUNK)], q_v[nb],
                        sem_s[nb]).wait()

                @pl.when(k + 1 < NCHUNK)
                def _():
                    pltpu.async_copy(
                        dst_hbm.at[pl.ds(cbase(k + 1), CHUNK)], idx_v[nb], sem_l[nb])
                    pltpu.async_copy(
                        q_hbm.at[pl.ds(cbase(k + 1), CHUNK)], q_v[nb], sem_l[nb])
                    pltpu.async_copy(
                        d_hbm.at[pl.ds(cbase(k + 1), CHUNK)], d_v[nb], sem_l[nb])

                def row2(i, carry2):
                    for r in range(2):
                        for j in range(HID // 16):
                            sl = pl.ds(j * 16, 16)
                            q_v[b][i * 2 + r, sl] = jnp.maximum(
                                q_v[b][i * 2 + r, sl] + d_v[b][i * 2 + r, sl], 0.0)
                    return carry2

                lax.fori_loop(0, CHUNK // 2, row2, 0)

                pltpu.async_copy(
                    q_v[b], acc_sh.at[idx_v[b]], sem_s[b], add=True)
        return carry

    lax.fori_loop(0, (NCHUNK + 1) // 2, pair, 0)
    pltpu.make_async_copy(
        q_hbm.at[pl.ds(cbase(0), CHUNK)], q_v[0], sem_s[0]).wait()

    plsc.subcore_barrier()
    _readout_acc(acc_sh, out_hbm, c, s, row_off)


@functools.partial(
    pl.kernel,
    out_type=jax.ShapeDtypeStruct((E, HID), jnp.float32),
    mesh=_sc_mesh,
    scratch_types=(
        [pltpu.VMEM((GCH,), jnp.int32)] * 2
        + [pltpu.VMEM((GCH, HID), jnp.float32)] * 2
        + [pltpu.SemaphoreType.DMA] * 6
    ),
)
def _sc_gather(tab_hbm, src_hbm, out_hbm,
               i0, i1, t0, t1,
               si0, si1, sg0, sg1, so0, so1):
    c = lax.axis_index("c")
    s = lax.axis_index("s")
    base0 = (c * NS + s) * EPW
    idx_v = (i0, i1)
    t_v = (t0, t1)
    sem_i = (si0, si1)
    sem_g = (sg0, sg1)
    sem_o = (so0, so1)

    def cbase(k):
        return pl.multiple_of(base0 + k * GCH, 8)

    pltpu.async_copy(src_hbm.at[pl.ds(cbase(0), GCH)], i0, si0)

    def pair(g, carry):
        for b in (0, 1):
            k = 2 * g + b
            nb = 1 - b

            @pl.when(k < GNC)
            def _():
                pltpu.make_async_copy(
                    src_hbm.at[pl.ds(cbase(k), GCH)], idx_v[b], sem_i[b]).wait()

                @pl.when(k >= 2)
                def _():
                    pltpu.make_async_copy(
                        t_v[b], out_hbm.at[pl.ds(cbase(k - 2), GCH)],
                        sem_o[b]).wait()

                cps = [
                    pltpu.async_copy(
                        tab_hbm.at[idx_v[b].at[pl.ds(j * 80, 80)]],
                        t_v[b].at[pl.ds(j * 80, 80)], sem_g[b])
                    for j in range(GCH // 80)
                ]

                @pl.when(k + 1 < GNC)
                def _():
                    pltpu.async_copy(
                        src_hbm.at[pl.ds(cbase(k + 1), GCH)], idx_v[nb], sem_i[nb])

                @pl.when(k >= 1)
                def _():
                    pltpu.async_copy(
                        t_v[nb], out_hbm.at[pl.ds(cbase(k - 1), GCH)], sem_o[nb])

                for cp in cps:
                    cp.wait()

            @pl.when(k == GNC)
            def _():
                pltpu.async_copy(
                    t_v[nb], out_hbm.at[pl.ds(cbase(GNC - 1), GCH)], sem_o[nb])
        return carry

    lax.fori_loop(0, (GNC + 2) // 2, pair, 0)
    pltpu.make_async_copy(
        t_v[1], out_hbm.at[pl.ds(cbase(GNC - 2), GCH)], sem_o[1]).wait()
    pltpu.make_async_copy(
        t_v[0], out_hbm.at[pl.ds(cbase(GNC - 1), GCH)], sem_o[0]).wait()


_SUBG = ((0, 80), (80, 80), (160, 40))


@functools.partial(
    pl.kernel,
    out_type=jax.ShapeDtypeStruct((E, HID), jnp.float32),
    mesh=_sc_mesh,
    scratch_types=(
        [pltpu.VMEM((SCH,), jnp.int32)] * 4
        + [pltpu.VMEM((SCH, HID), jnp.float32)] * 4
        + [pltpu.SemaphoreType.DMA] * 6
    ),
)
def _sc_gather_sub(tab_hbm, g_hbm, src_hbm, rev_hbm, out_hbm,
                   a0, a1, r0, r1, t0, t1, g0, g1,
                   si0, si1, sg0, sg1, so0, so1):
    c = lax.axis_index("c")
    s = lax.axis_index("s")
    base0 = (c * NS + s) * EPW
    sidx_v = (a0, a1)
    ridx_v = (r0, r1)
    t_v = (t0, t1)
    g_v = (g0, g1)
    sem_i = (si0, si1)
    sem_g = (sg0, sg1)
    sem_o = (so0, so1)

    def cbase(k):
        return pl.multiple_of(base0 + k * SCH, 8)

    pltpu.async_copy(src_hbm.at[pl.ds(cbase(0), SCH)], a0, si0)
    pltpu.async_copy(rev_hbm.at[pl.ds(cbase(0), SCH)], r0, si0)

    def pair(g, carry):
        for b in (0, 1):
            k = 2 * g + b
            nb = 1 - b

            @pl.when(k < SNC)
            def _():
                pltpu.make_async_copy(
                    src_hbm.at[pl.ds(cbase(k), SCH)], sidx_v[b], sem_i[b]).wait()
                pltpu.make_async_copy(
                    rev_hbm.at[pl.ds(cbase(k), SCH)], ridx_v[b], sem_i[b]).wait()

                @pl.when(k >= 2)
                def _():
                    pltpu.make_async_copy(
                        t_v[b], out_hbm.at[pl.ds(cbase(k - 2), SCH)],
                        sem_o[b]).wait()

                cps = []
                for off, ln in _SUBG:
                    cps.append(pltpu.async_copy(
                        tab_hbm.at[sidx_v[b].at[pl.ds(off, ln)]],
                        t_v[b].at[pl.ds(off, ln)], sem_g[b]))
                    cps.append(pltpu.async_copy(
                        g_hbm.at[ridx_v[b].at[pl.ds(off, ln)]],
                        g_v[b].at[pl.ds(off, ln)], sem_g[b]))

                @pl.when(k + 1 < SNC)
                def _():
                    pltpu.async_copy(
                        src_hbm.at[pl.ds(cbase(k + 1), SCH)], sidx_v[nb], sem_i[nb])
                    pltpu.async_copy(
                        rev_hbm.at[pl.ds(cbase(k + 1), SCH)], ridx_v[nb], sem_i[nb])

                @pl.when(k >= 1)
                def _():
                    def row2(i, carry2):
                        for r in range(2):
                            for j in range(HID // 16):
                                sl = pl.ds(j * 16, 16)
                                t_v[nb][i * 2 + r, sl] = (
                                    t_v[nb][i * 2 + r, sl]
                                    - g_v[nb][i * 2 + r, sl])
                        return carry2

                    lax.fori_loop(0, SCH // 2, row2, 0)
                    pltpu.async_copy(
                        t_v[nb], out_hbm.at[pl.ds(cbase(k - 1), SCH)], sem_o[nb])

                for cp in cps:
                    cp.wait()

            @pl.when(k == SNC)
            def _():
                def row2(i, carry2):
                    for r in range(2):
                        for j in range(HID // 16):
                            sl = pl.ds(j * 16, 16)
                            t_v[nb][i * 2 + r, sl] = (
                                t_v[nb][i * 2 + r, sl] - g_v[nb][i * 2 + r, sl])
                    return carry2

                lax.fori_loop(0, SCH // 2, row2, 0)
                pltpu.async_copy(
                    t_v[nb], out_hbm.at[pl.ds(cbase(SNC - 1), SCH)], sem_o[nb])
        return carry

    lax.fori_loop(0, (SNC + 2) // 2, pair, 0)
    pltpu.make_async_copy(
        t_v[1], out_hbm.at[pl.ds(cbase(SNC - 2), SCH)], sem_o[1]).wait()
    pltpu.make_async_copy(
        t_v[0], out_hbm.at[pl.ds(cbase(SNC - 1), SCH)], sem_o[0]).wait()


def kernel(x, edge_index, edge_attr, rev_edge_index, W_i, b_i, W_h, b_h, W_o, b_o):
    src = edge_index[0]
    dst = edge_index[1]
    wxt = jnp.transpose(W_i[:, :D_NODE_DIM])
    wet = jnp.transpose(W_i[:, D_NODE_DIM:])
    wht = jnp.transpose(W_h)
    wo1t = jnp.transpose(W_o[:, :D_NODE_DIM])
    wo2t = jnp.transpose(W_o[:, D_NODE_DIM:])
    bi2 = b_i.reshape(1, HID)
    bh2 = b_h.reshape(1, HID)
    bo2 = b_o.reshape(1, HID)

    p = _tc_matmul(x, wxt, block=2000)
    q = _tc_matmul(edge_attr, wet, bias=bi2)
    d = _sc_gather(p, src)
    g = _mm_relu_add(q, d, wht)
    for t in range(2):
        ab = _sc_scatter(g, dst)
        tt = _combine(ab, p, bh2)
        d = _sc_gather_sub(tt, g, src, rev_edge_index)
        if t == 0:
            g = _mm_relu_add(q, d, wht)
    ab = _sc_scatter_relu(q, d, dst)
    return _final(x, ab, wo1t, wo2t, bo2)


# speedup vs baseline: 1.0176x; 1.0020x over previous
import functools

import jax
import jax.numpy as jnp
from jax import lax
from jax.experimental import pallas as pl
from jax.experimental.pallas import tpu as pltpu
from jax.experimental.pallas import tpu_sc as plsc

N = 10000
E = 320000
D_NODE_DIM = 128
HID = 128
NC = 2
NS = 16
NW = NC * NS
EPW = E // NW
CHUNK = 80
NCHUNK = EPW // CHUNK
GCH = 400
GNC = EPW // GCH
SCH = 200
SNC = EPW // SCH
ROWS_A = 632
ROWS_B = N - (NS - 1) * ROWS_A

_sc_mesh = plsc.VectorSubcoreMesh(core_axis_name="c", subcore_axis_name="s")


def _mm_bias_body(a_ref, w_ref, b_ref, o_ref):
    o_ref[...] = (
        jnp.dot(a_ref[...], w_ref[...], preferred_element_type=jnp.float32)
        + b_ref[...]
    )


def _mm_body(a_ref, w_ref, o_ref):
    o_ref[...] = jnp.dot(a_ref[...], w_ref[...], preferred_element_type=jnp.float32)


def _tc_matmul(a, w, bias=None, block=16000):
    m, k = a.shape
    n = w.shape[1]
    grid = (m // block,)
    in_specs = [
        pl.BlockSpec((block, k), lambda i: (i, 0)),
        pl.BlockSpec((k, n), lambda i: (0, 0)),
    ]
    args = [a, w]
    body = _mm_body
    if bias is not None:
        in_specs.append(pl.BlockSpec((1, n), lambda i: (0, 0)))
        args.append(bias)
        body = _mm_bias_body
    return pl.pallas_call(
        body,
        grid=grid,
        in_specs=in_specs,
        out_specs=pl.BlockSpec((block, n), lambda i: (i, 0)),
        out_shape=jax.ShapeDtypeStruct((m, n), jnp.float32),
    )(*args)


def _mm_relu_add_body(q_ref, d_ref, w_ref, o_ref):
    h = jnp.maximum(q_ref[...] + d_ref[...], 0.0)
    o_ref[...] = jnp.dot(h, w_ref[...], preferred_element_type=jnp.float32)


def _mm_relu_add(q, d, w, block=16000):
    m = q.shape[0]
    n = w.shape[1]
    grid = (m // block,)
    return pl.pallas_call(
        _mm_relu_add_body,
        grid=grid,
        in_specs=[
            pl.BlockSpec((block, HID), lambda i: (i, 0)),
            pl.BlockSpec((block, HID), lambda i: (i, 0)),
            pl.BlockSpec((HID, n), lambda i: (0, 0)),
        ],
        out_specs=pl.BlockSpec((block, n), lambda i: (i, 0)),
        out_shape=jax.ShapeDtypeStruct((m, n), jnp.float32),
    )(q, d, w)


def _combine_body(ab_ref, p_ref, bh_ref, t_ref):
    t_ref[...] = ab_ref[0] + ab_ref[1] + p_ref[...] + bh_ref[...]


def _combine(ab, p, bh, block=5000):
    grid = (N // block,)
    return pl.pallas_call(
        _combine_body,
        grid=grid,
        in_specs=[
            pl.BlockSpec((NC, block, HID), lambda i: (0, i, 0)),
            pl.BlockSpec((block, HID), lambda i: (i, 0)),
            pl.BlockSpec((1, HID), lambda i: (0, 0)),
        ],
        out_specs=pl.BlockSpec((block, HID), lambda i: (i, 0)),
        out_shape=jax.ShapeDtypeStruct((N, HID), jnp.float32),
    )(ab, p, bh)


def _final_body(x_ref, ab_ref, w1_ref, w2_ref, b_ref, o_ref):
    f = ab_ref[0] + ab_ref[1]
    cond = jnp.sum(f, axis=1, keepdims=True) == 0.0
    mp = jnp.where(cond, x_ref[...], f)
    o_ref[...] = jax.nn.relu(
        jnp.dot(x_ref[...], w1_ref[...], preferred_element_type=jnp.float32)
        + jnp.dot(mp, w2_ref[...], preferred_element_type=jnp.float32)
        + b_ref[...]
    )


def _final(x, ab, w1t, w2t, bo, block=5000):
    grid = (N // block,)
    return pl.pallas_call(
        _final_body,
        grid=grid,
        in_specs=[
            pl.BlockSpec((block, D_NODE_DIM), lambda i: (i, 0)),
            pl.BlockSpec((NC, block, HID), lambda i: (0, i, 0)),
            pl.BlockSpec((D_NODE_DIM, HID), lambda i: (0, 0)),
            pl.BlockSpec((HID, HID), lambda i: (0, 0)),
            pl.BlockSpec((1, HID), lambda i: (0, 0)),
        ],
        out_specs=pl.BlockSpec((block, HID), lambda i: (i, 0)),
        out_shape=jax.ShapeDtypeStruct((N, HID), jnp.float32),
    )(x, ab, w1t, w2t, bo)


def _zero_acc(zbuf, acc_sh, s, row_off):
    def zrow(i, carry):
        for j in range(HID // 16):
            zbuf[i, pl.ds(j * 16, 16)] = jnp.zeros((16,), jnp.float32)
        return carry

    lax.fori_loop(0, CHUNK, zrow, 0)

    @pl.when(s < NS - 1)
    def _():
        def zfill(i, carry):
            off = pl.multiple_of(s * ROWS_A + i * CHUNK, 8)
            pltpu.sync_copy(zbuf, acc_sh.at[pl.ds(off, CHUNK)])
            return carry
        lax.fori_loop(0, ROWS_A // CHUNK, zfill, 0)
        pltpu.sync_copy(zbuf.at[pl.ds(0, ROWS_A % CHUNK)],
                        acc_sh.at[pl.ds(pl.multiple_of(
                            s * ROWS_A + (ROWS_A // CHUNK) * CHUNK, 8),
                            ROWS_A % CHUNK)])

    @pl.when(s == NS - 1)
    def _():
        base_b = (NS - 1) * ROWS_A

        def zfill(i, carry):
            off = pl.multiple_of(base_b + i * CHUNK, 8)
            pltpu.sync_copy(zbuf, acc_sh.at[pl.ds(off, CHUNK)])
            return carry
        lax.fori_loop(0, ROWS_B // CHUNK, zfill, 0)
        pltpu.sync_copy(zbuf.at[pl.ds(0, ROWS_B % CHUNK)],
                        acc_sh.at[pl.ds(base_b + (ROWS_B // CHUNK) * CHUNK,
                                        ROWS_B % CHUNK)])


def _readout_acc(acc_sh, out_hbm, c, s, row_off):
    @pl.when(s < NS - 1)
    def _():
        pltpu.sync_copy(acc_sh.at[pl.ds(row_off, ROWS_A)],
                        out_hbm.at[c, pl.ds(row_off, ROWS_A)])

    @pl.when(s == NS - 1)
    def _():
        pltpu.sync_copy(acc_sh.at[pl.ds((NS - 1) * ROWS_A, ROWS_B)],
                        out_hbm.at[c, pl.ds((NS - 1) * ROWS_A, ROWS_B)])


@functools.partial(
    pl.kernel,
    out_type=jax.ShapeDtypeStruct((NC, N, HID), jnp.float32),
    mesh=_sc_mesh,
    scratch_types=[
        pltpu.VMEM((CHUNK,), jnp.int32),
        pltpu.VMEM((CHUNK,), jnp.int32),
        pltpu.VMEM((CHUNK, HID), jnp.float32),
        pltpu.VMEM((CHUNK, HID), jnp.float32),
        pltpu.VMEM_SHARED((N, HID), jnp.float32),
        pltpu.SemaphoreType.DMA,
        pltpu.SemaphoreType.DMA,
        pltpu.SemaphoreType.DMA,
        pltpu.SemaphoreType.DMA,
    ],
)
def _sc_scatter(rows_hbm, dst_hbm, out_hbm,
                idx0, idx1, rows0, rows1, acc_sh,
                sem_l0, sem_l1, sem_s0, sem_s1):
    c = lax.axis_index("c")
    s = lax.axis_index("s")
    wid = c * NS + s
    row_off = pl.multiple_of(s * ROWS_A, 8)
    idx_v = (idx0, idx1)
    rows_v = (rows0, rows1)
    sem_l = (sem_l0, sem_l1)
    sem_s = (sem_s0, sem_s1)

    _zero_acc(rows0, acc_sh, s, row_off)
    plsc.subcore_barrier()

    base0 = wid * EPW

    def cbase(k):
        return pl.multiple_of(base0 + k * CHUNK, 8)

    pltpu.async_copy(dst_hbm.at[pl.ds(cbase(0), CHUNK)], idx0, sem_l0)
    pltpu.async_copy(rows_hbm.at[pl.ds(cbase(0), CHUNK)], rows0, sem_l0)

    def pair(g, carry):
        for b in (0, 1):
            k = 2 * g + b
            nb = 1 - b

            @pl.when(k < NCHUNK)
            def _():
                pltpu.make_async_copy(
                    dst_hbm.at[pl.ds(cbase(k), CHUNK)], idx_v[b], sem_l[b]).wait()
                pltpu.make_async_copy(
                    rows_hbm.at[pl.ds(cbase(k), CHUNK)], rows_v[b], sem_l[b]).wait()

                @pl.when(k >= 1)
                def _():
                    pltpu.make_async_copy(
                        rows_hbm.at[pl.ds(cbase(0), CHUNK)], rows_v[nb],
                        sem_s[nb]).wait()

                @pl.when(k + 1 < NCHUNK)
                def _():
                    pltpu.async_copy(
                        dst_hbm.at[pl.ds(cbase(k + 1), CHUNK)], idx_v[nb], sem_l[nb])
                    pltpu.async_copy(
                        rows_hbm.at[pl.ds(cbase(k + 1), CHUNK)], rows_v[nb], sem_l[nb])

                pltpu.async_copy(
                    rows_v[b], acc_sh.at[idx_v[b]], sem_s[b], add=True)
        return carry

    lax.fori_loop(0, (NCHUNK + 1) // 2, pair, 0)
    pltpu.make_async_copy(
        rows_hbm.at[pl.ds(cbase(0), CHUNK)], rows_v[0], sem_s[0]).wait()

    plsc.subcore_barrier()
    _readout_acc(acc_sh, out_hbm, c, s, row_off)


@functools.partial(
    pl.kernel,
    out_type=jax.ShapeDtypeStruct((NC, N, HID), jnp.float32),
    mesh=_sc_mesh,
    scratch_types=[
        pltpu.VMEM((CHUNK,), jnp.int32),
        pltpu.VMEM((CHUNK,), jnp.int32),
        pltpu.VMEM((CHUNK, HID), jnp.float32),
        pltpu.VMEM((CHUNK, HID), jnp.float32),
        pltpu.VMEM((CHUNK, HID), jnp.float32),
        pltpu.VMEM((CHUNK, HID), jnp.float32),
        pltpu.VMEM_SHARED((N, HID), jnp.float32),
        pltpu.SemaphoreType.DMA,
        pltpu.SemaphoreType.DMA,
        pltpu.SemaphoreType.DMA,
        pltpu.SemaphoreType.DMA,
    ],
)
def _sc_scatter_relu(q_hbm, d_hbm, dst_hbm, out_hbm,
                     idx0, idx1, q0, q1, d0, d1, acc_sh,
                     sem_l0, sem_l1, sem_s0, sem_s1):
    c = lax.axis_index("c")
    s = lax.axis_index("s")
    wid = c * NS + s
    row_off = pl.multiple_of(s * ROWS_A, 8)
    idx_v = (idx0, idx1)
    q_v = (q0, q1)
    d_v = (d0, d1)
    sem_l = (sem_l0, sem_l1)
    sem_s = (sem_s0, sem_s1)

    _zero_acc(q0, acc_sh, s, row_off)
    plsc.subcore_barrier()

    base0 = wid * EPW

    def cbase(k):
        return pl.multiple_of(base0 + k * CHUNK, 8)

    pltpu.async_copy(dst_hbm.at[pl.ds(cbase(0), CHUNK)], idx0, sem_l0)
    pltpu.async_copy(q_hbm.at[pl.ds(cbase(0), CHUNK)], q0, sem_l0)
    pltpu.async_copy(d_hbm.at[pl.ds(cbase(0), CHUNK)], d0, sem_l0)

    def pair(g, carry):
        for b in (0, 1):
            k = 2 * g + b
            nb = 1 - b

            @pl.when(k < NCHUNK)
            def _():
                pltpu.make_async_copy(
                    dst_hbm.at[pl.ds(cbase(k), CHUNK)], idx_v[b], sem_l[b]).wait()
                pltpu.make_async_copy(
                    q_hbm.at[pl.ds(cbase(k), CHUNK)], q_v[b], sem_l[b]).wait()
                pltpu.make_async_copy(
                    d_hbm.at[pl.ds(cbase(k), CHUNK)], d_v[b], sem_l[b]).wait()

                @pl.when(k >= 1)
                def _():
                    pltpu.make_async_copy(
                        q_hbm.at[pl.ds(cbase(0), CHUNK)], q_v[nb],
                        sem_s[nb]).wait()

                @pl.when(k + 1 < NCHUNK)
                def _():
                    pltpu.async_copy(
                        dst_hbm.at[pl.ds(cbase(k + 1), CHUNK)], idx_v[nb], sem_l[nb])
                    pltpu.async_copy(
                        q_hbm.at[pl.ds(cbase(k + 1), CHUNK)], q_v[nb], sem_l[nb])
                    pltpu.async_copy(
                        d_hbm.at[pl.ds(cbase(k + 1), CHUNK)], d_v[nb], sem_l[nb])

                def row2(i, carry2):
                    for r in range(2):
                        for j in range(HID // 16):
                            sl = pl.ds(j * 16, 16)
                            q_v[b][i * 2 + r, sl] = jnp.maximum(
                                q_v[b][i * 2 + r, sl] + d_v[b][i * 2 + r, sl], 0.0)
                    return carry2

                lax.fori_loop(0, CHUNK // 2, row2, 0)

                pltpu.async_copy(
                    q_v[b], acc_sh.at[idx_v[b]], sem_s[b], add=True)
        return carry

    lax.fori_loop(0, (NCHUNK + 1) // 2, pair, 0)
    pltpu.make_async_copy(
        q_hbm.at[pl.ds(cbase(0), CHUNK)], q_v[0], sem_s[0]).wait()

    plsc.subcore_barrier()
    _readout_acc(acc_sh, out_hbm, c, s, row_off)


@functools.partial(
    pl.kernel,
    out_type=jax.ShapeDtypeStruct((E, HID), jnp.float32),
    mesh=_sc_mesh,
    scratch_types=(
        [pltpu.VMEM((GCH,), jnp.int32)] * 2
        + [pltpu.VMEM((GCH, HID), jnp.float32)] * 2
        + [pltpu.SemaphoreType.DMA] * 6
    ),
)
def _sc_gather(tab_hbm, src_hbm, out_hbm,
               i0, i1, t0, t1,
               si0, si1, sg0, sg1, so0, so1):
    c = lax.axis_index("c")
    s = lax.axis_index("s")
    base0 = (c * NS + s) * EPW
    idx_v = (i0, i1)
    t_v = (t0, t1)
    sem_i = (si0, si1)
    sem_g = (sg0, sg1)
    sem_o = (so0, so1)

    def cbase(k):
        return pl.multiple_of(base0 + k * GCH, 8)

    pltpu.async_copy(src_hbm.at[pl.ds(cbase(0), GCH)], i0, si0)

    def pair(g, carry):
        for b in (0, 1):
            k = 2 * g + b
            nb = 1 - b

            @pl.when(k < GNC)
            def _():
                pltpu.make_async_copy(
                    src_hbm.at[pl.ds(cbase(k), GCH)], idx_v[b], sem_i[b]).wait()

                @pl.when(k >= 2)
                def _():
                    pltpu.make_async_copy(
                        t_v[b], out_hbm.at[pl.ds(cbase(k - 2), GCH)],
                        sem_o[b]).wait()

                cps = [
                    pltpu.async_copy(
                        tab_hbm.at[idx_v[b].at[pl.ds(j * 80, 80)]],
                        t_v[b].at[pl.ds(j * 80, 80)], sem_g[b])
                    for j in range(GCH // 80)
                ]

                @pl.when(k + 1 < GNC)
                def _():
                    pltpu.async_copy(
                        src_hbm.at[pl.ds(cbase(k + 1), GCH)], idx_v[nb], sem_i[nb])

                @pl.when(k >= 1)
                def _():
                    pltpu.async_copy(
                        t_v[nb], out_hbm.at[pl.ds(cbase(k - 1), GCH)], sem_o[nb])

                for cp in cps:
                    cp.wait()

            @pl.when(k == GNC)
            def _():
                pltpu.async_copy(
                    t_v[nb], out_hbm.at[pl.ds(cbase(GNC - 1), GCH)], sem_o[nb])
        return carry

    lax.fori_loop(0, (GNC + 2) // 2, pair, 0)
    pltpu.make_async_copy(
        t_v[1], out_hbm.at[pl.ds(cbase(GNC - 2), GCH)], sem_o[1]).wait()
    pltpu.make_async_copy(
        t_v[0], out_hbm.at[pl.ds(cbase(GNC - 1), GCH)], sem_o[0]).wait()


_SUBG = ((0, 80), (80, 80), (160, 40))


@functools.partial(
    pl.kernel,
    out_type=jax.ShapeDtypeStruct((E, HID), jnp.float32),
    mesh=_sc_mesh,
    scratch_types=(
        [pltpu.VMEM((SCH,), jnp.int32)] * 4
        + [pltpu.VMEM((SCH, HID), jnp.float32)] * 4
        + [pltpu.SemaphoreType.DMA] * 6
    ),
)
def _sc_gather_sub(tab_hbm, g_hbm, src_hbm, rev_hbm, out_hbm,
                   a0, a1, r0, r1, t0, t1, g0, g1,
                   si0, si1, sg0, sg1, so0, so1):
    c = lax.axis_index("c")
    s = lax.axis_index("s")
    base0 = (c * NS + s) * EPW
    sidx_v = (a0, a1)
    ridx_v = (r0, r1)
    t_v = (t0, t1)
    g_v = (g0, g1)
    sem_i = (si0, si1)
    sem_g = (sg0, sg1)
    sem_o = (so0, so1)

    def cbase(k):
        return pl.multiple_of(base0 + k * SCH, 8)

    pltpu.async_copy(src_hbm.at[pl.ds(cbase(0), SCH)], a0, si0)
    pltpu.async_copy(rev_hbm.at[pl.ds(cbase(0), SCH)], r0, si0)

    def pair(g, carry):
        for b in (0, 1):
            k = 2 * g + b
            nb = 1 - b

            @pl.when(k < SNC)
            def _():
                pltpu.make_async_copy(
                    src_hbm.at[pl.ds(cbase(k), SCH)], sidx_v[b], sem_i[b]).wait()
                pltpu.make_async_copy(
                    rev_hbm.at[pl.ds(cbase(k), SCH)], ridx_v[b], sem_i[b]).wait()

                @pl.when(k >= 2)
                def _():
                    pltpu.make_async_copy(
                        t_v[b], out_hbm.at[pl.ds(cbase(k - 2), SCH)],
                        sem_o[b]).wait()

                cps = []
                for off, ln in _SUBG:
                    cps.append(pltpu.async_copy(
                        tab_hbm.at[sidx_v[b].at[pl.ds(off, ln)]],
                        t_v[b].at[pl.ds(off, ln)], sem_g[b]))
                    cps.append(pltpu.async_copy(
                        g_hbm.at[ridx_v[b].at[pl.ds(off, ln)]],
                        g_v[b].at[pl.ds(off, ln)], sem_g[b]))

                @pl.when(k + 1 < SNC)
                def _():
                    pltpu.async_copy(
                        src_hbm.at[pl.ds(cbase(k + 1), SCH)], sidx_v[nb], sem_i[nb])
                    pltpu.async_copy(
                        rev_hbm.at[pl.ds(cbase(k + 1), SCH)], ridx_v[nb], sem_i[nb])

                @pl.when(k >= 1)
                def _():
                    def row2(i, carry2):
                        for r in range(2):
                            for j in range(HID // 16):
                                sl = pl.ds(j * 16, 16)
                                t_v[nb][i * 2 + r, sl] = (
                                    t_v[nb][i * 2 + r, sl]
                                    - g_v[nb][i * 2 + r, sl])
                        return carry2

                    lax.fori_loop(0, SCH // 2, row2, 0)
                    pltpu.async_copy(
                        t_v[nb], out_hbm.at[pl.ds(cbase(k - 1), SCH)], sem_o[nb])

                for cp in cps:
                    cp.wait()

            @pl.when(k == SNC)
            def _():
                def row2(i, carry2):
                    for r in range(2):
                        for j in range(HID // 16):
                            sl = pl.ds(j * 16, 16)
                            t_v[nb][i * 2 + r, sl] = (
                                t_v[nb][i * 2 + r, sl] - g_v[nb][i * 2 + r, sl])
                    return carry2

                lax.fori_loop(0, SCH // 2, row2, 0)
                pltpu.async_copy(
                    t_v[nb], out_hbm.at[pl.ds(cbase(SNC - 1), SCH)], sem_o[nb])
        return carry

    lax.fori_loop(0, (SNC + 2) // 2, pair, 0)
    pltpu.make_async_copy(
        t_v[1], out_hbm.at[pl.ds(cbase(SNC - 2), SCH)], sem_o[1]).wait()
    pltpu.make_async_copy(
        t_v[0], out_hbm.at[pl.ds(cbase(SNC - 1), SCH)], sem_o[0]).wait()


def kernel(x, edge_index, edge_attr, rev_edge_index, W_i, b_i, W_h, b_h, W_o, b_o):
    src = edge_index[0]
    dst = edge_index[1]
    wxt = jnp.transpose(W_i[:, :D_NODE_DIM])
    wet = jnp.transpose(W_i[:, D_NODE_DIM:])
    wht = jnp.transpose(W_h)
    wo1t = jnp.transpose(W_o[:, :D_NODE_DIM])
    wo2t = jnp.transpose(W_o[:, D_NODE_DIM:])
    bi2 = b_i.reshape(1, HID)
    bh2 = b_h.reshape(1, HID)
    bo2 = b_o.reshape(1, HID)

    p = _tc_matmul(x, wxt, block=2000)
    q = _tc_matmul(edge_attr, wet, bias=bi2)
    d = _sc_gather(p, src)
    g = _mm_relu_add(q, d, wht)
    for t in range(2):
        ab = _sc_scatter(g, dst)
        tt = _combine(ab, p, bh2)
        d = _sc_gather_sub(tt, g, src, rev_edge_index)
        if t == 0:
            g = _mm_relu_add(q, d, wht)
    ab = _sc_scatter_relu(q, d, dst)
    return _final(x, ab, wo1t, wo2t, bo2)


# speedup vs baseline: 1.0177x; 1.0000x over previous
import functools

import jax
import jax.numpy as jnp
from jax import lax
from jax.experimental import pallas as pl
from jax.experimental.pallas import tpu as pltpu
from jax.experimental.pallas import tpu_sc as plsc

N = 10000
E = 320000
D_NODE_DIM = 128
HID = 128
NC = 2
NS = 16
NW = NC * NS
EPW = E // NW
CHUNK = 80
NCHUNK = EPW // CHUNK
GCH = 400
GNC = EPW // GCH
SCH = 200
SNC = EPW // SCH
ROWS_A = 632
ROWS_B = N - (NS - 1) * ROWS_A

_sc_mesh = plsc.VectorSubcoreMesh(core_axis_name="c", subcore_axis_name="s")


def _mm_bias_body(a_ref, w_ref, b_ref, o_ref):
    o_ref[...] = (
        jnp.dot(a_ref[...], w_ref[...], preferred_element_type=jnp.float32)
        + b_ref[...]
    )


def _mm_body(a_ref, w_ref, o_ref):
    o_ref[...] = jnp.dot(a_ref[...], w_ref[...], preferred_element_type=jnp.float32)


def _tc_matmul(a, w, bias=None, block=16000):
    m, k = a.shape
    n = w.shape[1]
    grid = (m // block,)
    in_specs = [
        pl.BlockSpec((block, k), lambda i: (i, 0)),
        pl.BlockSpec((k, n), lambda i: (0, 0)),
    ]
    args = [a, w]
    body = _mm_body
    if bias is not None:
        in_specs.append(pl.BlockSpec((1, n), lambda i: (0, 0)))
        args.append(bias)
        body = _mm_bias_body
    return pl.pallas_call(
        body,
        grid=grid,
        in_specs=in_specs,
        out_specs=pl.BlockSpec((block, n), lambda i: (i, 0)),
        out_shape=jax.ShapeDtypeStruct((m, n), jnp.float32),
    )(*args)


def _mm_relu_add_body(q_ref, d_ref, w_ref, o_ref):
    h = jnp.maximum(q_ref[...] + d_ref[...], 0.0)
    o_ref[...] = jnp.dot(h, w_ref[...], preferred_element_type=jnp.float32)


def _mm_relu_add(q, d, w, block=16000):
    m = q.shape[0]
    n = w.shape[1]
    grid = (m // block,)
    return pl.pallas_call(
        _mm_relu_add_body,
        grid=grid,
        in_specs=[
            pl.BlockSpec((block, HID), lambda i: (i, 0)),
            pl.BlockSpec((block, HID), lambda i: (i, 0)),
            pl.BlockSpec((HID, n), lambda i: (0, 0)),
        ],
        out_specs=pl.BlockSpec((block, n), lambda i: (i, 0)),
        out_shape=jax.ShapeDtypeStruct((m, n), jnp.float32),
    )(q, d, w)


def _combine_body(ab_ref, p_ref, bh_ref, t_ref):
    t_ref[...] = ab_ref[0] + ab_ref[1] + p_ref[...] + bh_ref[...]


def _combine(ab, p, bh, block=5000):
    grid = (N // block,)
    return pl.pallas_call(
        _combine_body,
        grid=grid,
        in_specs=[
            pl.BlockSpec((NC, block, HID), lambda i: (0, i, 0)),
            pl.BlockSpec((block, HID), lambda i: (i, 0)),
            pl.BlockSpec((1, HID), lambda i: (0, 0)),
        ],
        out_specs=pl.BlockSpec((block, HID), lambda i: (i, 0)),
        out_shape=jax.ShapeDtypeStruct((N, HID), jnp.float32),
    )(ab, p, bh)


def _final_body(x_ref, ab_ref, w1_ref, w2_ref, b_ref, o_ref):
    f = ab_ref[0] + ab_ref[1]
    cond = jnp.sum(f, axis=1, keepdims=True) == 0.0
    mp = jnp.where(cond, x_ref[...], f)
    o_ref[...] = jax.nn.relu(
        jnp.dot(x_ref[...], w1_ref[...], preferred_element_type=jnp.float32)
        + jnp.dot(mp, w2_ref[...], preferred_element_type=jnp.float32)
        + b_ref[...]
    )


def _final(x, ab, w1t, w2t, bo, block=5000):
    grid = (N // block,)
    return pl.pallas_call(
        _final_body,
        grid=grid,
        in_specs=[
            pl.BlockSpec((block, D_NODE_DIM), lambda i: (i, 0)),
            pl.BlockSpec((NC, block, HID), lambda i: (0, i, 0)),
            pl.BlockSpec((D_NODE_DIM, HID), lambda i: (0, 0)),
            pl.BlockSpec((HID, HID), lambda i: (0, 0)),
            pl.BlockSpec((1, HID), lambda i: (0, 0)),
        ],
        out_specs=pl.BlockSpec((block, HID), lambda i: (i, 0)),
        out_shape=jax.ShapeDtypeStruct((N, HID), jnp.float32),
    )(x, ab, w1t, w2t, bo)


def _zero_acc(zbuf, acc_sh, s, row_off, zsem):
    def zrow(i, carry):
        for j in range(HID // 16):
            zbuf[i, pl.ds(j * 16, 16)] = jnp.zeros((16,), jnp.float32)
        return carry

    lax.fori_loop(0, CHUNK, zrow, 0)

    @pl.when(s < NS - 1)
    def _():
        def zfill(i, carry):
            off = pl.multiple_of(s * ROWS_A + i * CHUNK, 8)
            pltpu.async_copy(zbuf, acc_sh.at[pl.ds(off, CHUNK)], zsem)
            return carry
        lax.fori_loop(0, ROWS_A // CHUNK, zfill, 0)
        pltpu.async_copy(zbuf.at[pl.ds(0, ROWS_A % CHUNK)],
                         acc_sh.at[pl.ds(pl.multiple_of(
                             s * ROWS_A + (ROWS_A // CHUNK) * CHUNK, 8),
                             ROWS_A % CHUNK)], zsem)
        def zdrain(i, carry):
            off = pl.multiple_of(s * ROWS_A + i * CHUNK, 8)
            pltpu.make_async_copy(
                zbuf, acc_sh.at[pl.ds(off, CHUNK)], zsem).wait()
            return carry
        lax.fori_loop(0, ROWS_A // CHUNK, zdrain, 0)
        pltpu.make_async_copy(
            zbuf.at[pl.ds(0, ROWS_A % CHUNK)],
            acc_sh.at[pl.ds(pl.multiple_of(
                s * ROWS_A + (ROWS_A // CHUNK) * CHUNK, 8),
                ROWS_A % CHUNK)], zsem).wait()

    @pl.when(s == NS - 1)
    def _():
        base_b = (NS - 1) * ROWS_A

        def zfill(i, carry):
            off = pl.multiple_of(base_b + i * CHUNK, 8)
            pltpu.async_copy(zbuf, acc_sh.at[pl.ds(off, CHUNK)], zsem)
            return carry
        lax.fori_loop(0, ROWS_B // CHUNK, zfill, 0)
        pltpu.async_copy(zbuf.at[pl.ds(0, ROWS_B % CHUNK)],
                         acc_sh.at[pl.ds(base_b + (ROWS_B // CHUNK) * CHUNK,
                                         ROWS_B % CHUNK)], zsem)
        def zdrain(i, carry):
            off = pl.multiple_of(base_b + i * CHUNK, 8)
            pltpu.make_async_copy(
                zbuf, acc_sh.at[pl.ds(off, CHUNK)], zsem).wait()
            return carry
        lax.fori_loop(0, ROWS_B // CHUNK, zdrain, 0)
        pltpu.make_async_copy(
            zbuf.at[pl.ds(0, ROWS_B % CHUNK)],
            acc_sh.at[pl.ds(base_b + (ROWS_B // CHUNK) * CHUNK,
                            ROWS_B % CHUNK)], zsem).wait()


def _readout_acc(acc_sh, out_hbm, c, s, row_off):
    @pl.when(s < NS - 1)
    def _():
        pltpu.sync_copy(acc_sh.at[pl.ds(row_off, ROWS_A)],
                        out_hbm.at[c, pl.ds(row_off, ROWS_A)])

    @pl.when(s == NS - 1)
    def _():
        pltpu.sync_copy(acc_sh.at[pl.ds((NS - 1) * ROWS_A, ROWS_B)],
                        out_hbm.at[c, pl.ds((NS - 1) * ROWS_A, ROWS_B)])


@functools.partial(
    pl.kernel,
    out_type=jax.ShapeDtypeStruct((NC, N, HID), jnp.float32),
    mesh=_sc_mesh,
    scratch_types=[
        pltpu.VMEM((CHUNK,), jnp.int32),
        pltpu.VMEM((CHUNK,), jnp.int32),
        pltpu.VMEM((CHUNK, HID), jnp.float32),
        pltpu.VMEM((CHUNK, HID), jnp.float32),
        pltpu.VMEM_SHARED((N, HID), jnp.float32),
        pltpu.SemaphoreType.DMA,
        pltpu.SemaphoreType.DMA,
        pltpu.SemaphoreType.DMA,
        pltpu.SemaphoreType.DMA,
    ],
)
def _sc_scatter(rows_hbm, dst_hbm, out_hbm,
                idx0, idx1, rows0, rows1, acc_sh,
                sem_l0, sem_l1, sem_s0, sem_s1):
    c = lax.axis_index("c")
    s = lax.axis_index("s")
    wid = c * NS + s
    row_off = pl.multiple_of(s * ROWS_A, 8)
    idx_v = (idx0, idx1)
    rows_v = (rows0, rows1)
    sem_l = (sem_l0, sem_l1)
    sem_s = (sem_s0, sem_s1)

    _zero_acc(rows0, acc_sh, s, row_off, sem_s0)
    plsc.subcore_barrier()

    base0 = wid * EPW

    def cbase(k):
        return pl.multiple_of(base0 + k * CHUNK, 8)

    pltpu.async_copy(dst_hbm.at[pl.ds(cbase(0), CHUNK)], idx0, sem_l0)
    pltpu.async_copy(rows_hbm.at[pl.ds(cbase(0), CHUNK)], rows0, sem_l0)

    def pair(g, carry):
        for b in (0, 1):
            k = 2 * g + b
            nb = 1 - b

            @pl.when(k < NCHUNK)
            def _():
                pltpu.make_async_copy(
                    dst_hbm.at[pl.ds(cbase(k), CHUNK)], idx_v[b], sem_l[b]).wait()
                pltpu.make_async_copy(
                    rows_hbm.at[pl.ds(cbase(k), CHUNK)], rows_v[b], sem_l[b]).wait()

                @pl.when(k >= 1)
                def _():
                    pltpu.make_async_copy(
                        rows_hbm.at[pl.ds(cbase(0), CHUNK)], rows_v[nb],
                        sem_s[nb]).wait()

                @pl.when(k + 1 < NCHUNK)
                def _():
                    pltpu.async_copy(
                        dst_hbm.at[pl.ds(cbase(k + 1), CHUNK)], idx_v[nb], sem_l[nb])
                    pltpu.async_copy(
                        rows_hbm.at[pl.ds(cbase(k + 1), CHUNK)], rows_v[nb], sem_l[nb])

                pltpu.async_copy(
                    rows_v[b], acc_sh.at[idx_v[b]], sem_s[b], add=True)
        return carry

    lax.fori_loop(0, (NCHUNK + 1) // 2, pair, 0)
    pltpu.make_async_copy(
        rows_hbm.at[pl.ds(cbase(0), CHUNK)], rows_v[0], sem_s[0]).wait()

    plsc.subcore_barrier()
    _readout_acc(acc_sh, out_hbm, c, s, row_off)


@functools.partial(
    pl.kernel,
    out_type=jax.ShapeDtypeStruct((NC, N, HID), jnp.float32),
    mesh=_sc_mesh,
    scratch_types=[
        pltpu.VMEM((CHUNK,), jnp.int32),
        pltpu.VMEM((CHUNK,), jnp.int32),
        pltpu.VMEM((CHUNK, HID), jnp.float32),
        pltpu.VMEM((CHUNK, HID), jnp.float32),
        pltpu.VMEM((CHUNK, HID), jnp.float32),
        pltpu.VMEM((CHUNK, HID), jnp.float32),
        pltpu.VMEM_SHARED((N, HID), jnp.float32),
        pltpu.SemaphoreType.DMA,
        pltpu.SemaphoreType.DMA,
        pltpu.SemaphoreType.DMA,
        pltpu.SemaphoreType.DMA,
    ],
)
def _sc_scatter_relu(q_hbm, d_hbm, dst_hbm, out_hbm,
                     idx0, idx1, q0, q1, d0, d1, acc_sh,
                     sem_l0, sem_l1, sem_s0, sem_s1):
    c = lax.axis_index("c")
    s = lax.axis_index("s")
    wid = c * NS + s
    row_off = pl.multiple_of(s * ROWS_A, 8)
    idx_v = (idx0, idx1)
    q_v = (q0, q1)
    d_v = (d0, d1)
    sem_l = (sem_l0, sem_l1)
    sem_s = (sem_s0, sem_s1)

    _zero_acc(q0, acc_sh, s, row_off, sem_s0)
    plsc.subcore_barrier()

    base0 = wid * EPW

    def cbase(k):
        return pl.multiple_of(base0 + k * CHUNK, 8)

    pltpu.async_copy(dst_hbm.at[pl.ds(cbase(0), CHUNK)], idx0, sem_l0)
    pltpu.async_copy(q_hbm.at[pl.ds(cbase(0), CHUNK)], q0, sem_l0)
    pltpu.async_copy(d_hbm.at[pl.ds(cbase(0), CHUNK)], d0, sem_l0)

    def pair(g, carry):
        for b in (0, 1):
            k = 2 * g + b
            nb = 1 - b

            @pl.when(k < NCHUNK)
            def _():
                pltpu.make_async_copy(
                    dst_hbm.at[pl.ds(cbase(k), CHUNK)], idx_v[b], sem_l[b]).wait()
                pltpu.make_async_copy(
                    q_hbm.at[pl.ds(cbase(k), CHUNK)], q_v[b], sem_l[b]).wait()
                pltpu.make_async_copy(
                    d_hbm.at[pl.ds(cbase(k), CHUNK)], d_v[b], sem_l[b]).wait()

                @pl.when(k >= 1)
                def _():
                    pltpu.make_async_copy(
                        q_hbm.at[pl.ds(cbase(0), CHUNK)], q_v[nb],
                        sem_s[nb]).wait()

                @pl.when(k + 1 < NCHUNK)
                def _():
                    pltpu.async_copy(
                        dst_hbm.at[pl.ds(cbase(k + 1), CHUNK)], idx_v[nb], sem_l[nb])
                    pltpu.async_copy(
                        q_hbm.at[pl.ds(cbase(k + 1), CHUNK)], q_v[nb], sem_l[nb])
                    pltpu.async_copy(
                        d_hbm.at[pl.ds(cbase(k + 1), CHUNK)], d_v[nb], sem_l[nb])

                def row2(i, carry2):
                    for r in range(2):
                        for j in range(HID // 16):
                            sl = pl.ds(j * 16, 16)
                            q_v[b][i * 2 + r, sl] = jnp.maximum(
                                q_v[b][i * 2 + r, sl] + d_v[b][i * 2 + r, sl], 0.0)
                    return carry2

                lax.fori_loop(0, CHUNK // 2, row2, 0)

                pltpu.async_copy(
                    q_v[b], acc_sh.at[idx_v[b]], sem_s[b], add=True)
        return carry

    lax.fori_loop(0, (NCHUNK + 1) // 2, pair, 0)
    pltpu.make_async_copy(
        q_hbm.at[pl.ds(cbase(0), CHUNK)], q_v[0], sem_s[0]).wait()

    plsc.subcore_barrier()
    _readout_acc(acc_sh, out_hbm, c, s, row_off)


@functools.partial(
    pl.kernel,
    out_type=jax.ShapeDtypeStruct((E, HID), jnp.float32),
    mesh=_sc_mesh,
    scratch_types=(
        [pltpu.VMEM((GCH,), jnp.int32)] * 2
        + [pltpu.VMEM((GCH, HID), jnp.float32)] * 2
        + [pltpu.SemaphoreType.DMA] * 6
    ),
)
def _sc_gather(tab_hbm, src_hbm, out_hbm,
               i0, i1, t0, t1,
               si0, si1, sg0, sg1, so0, so1):
    c = lax.axis_index("c")
    s = lax.axis_index("s")
    base0 = (c * NS + s) * EPW
    idx_v = (i0, i1)
    t_v = (t0, t1)
    sem_i = (si0, si1)
    sem_g = (sg0, sg1)
    sem_o = (so0, so1)

    def cbase(k):
        return pl.multiple_of(base0 + k * GCH, 8)

    pltpu.async_copy(src_hbm.at[pl.ds(cbase(0), GCH)], i0, si0)

    def pair(g, carry):
        for b in (0, 1):
            k = 2 * g + b
            nb = 1 - b

            @pl.when(k < GNC)
            def _():
                pltpu.make_async_copy(
                    src_hbm.at[pl.ds(cbase(k), GCH)], idx_v[b], sem_i[b]).wait()

                @pl.when(k >= 2)
                def _():
                    pltpu.make_async_copy(
                        t_v[b], out_hbm.at[pl.ds(cbase(k - 2), GCH)],
                        sem_o[b]).wait()

                cps = [
                    pltpu.async_copy(
                        tab_hbm.at[idx_v[b].at[pl.ds(j * 80, 80)]],
                        t_v[b].at[pl.ds(j * 80, 80)], sem_g[b])
                    for j in range(GCH // 80)
                ]

                @pl.when(k + 1 < GNC)
                def _():
                    pltpu.async_copy(
                        src_hbm.at[pl.ds(cbase(k + 1), GCH)], idx_v[nb], sem_i[nb])

                @pl.when(k >= 1)
                def _():
                    pltpu.async_copy(
                        t_v[nb], out_hbm.at[pl.ds(cbase(k - 1), GCH)], sem_o[nb])

                for cp in cps:
                    cp.wait()

            @pl.when(k == GNC)
            def _():
                pltpu.async_copy(
                    t_v[nb], out_hbm.at[pl.ds(cbase(GNC - 1), GCH)], sem_o[nb])
        return carry

    lax.fori_loop(0, (GNC + 2) // 2, pair, 0)
    pltpu.make_async_copy(
        t_v[1], out_hbm.at[pl.ds(cbase(GNC - 2), GCH)], sem_o[1]).wait()
    pltpu.make_async_copy(
        t_v[0], out_hbm.at[pl.ds(cbase(GNC - 1), GCH)], sem_o[0]).wait()


_SUBG = ((0, 80), (80, 80), (160, 40))


@functools.partial(
    pl.kernel,
    out_type=jax.ShapeDtypeStruct((E, HID), jnp.float32),
    mesh=_sc_mesh,
    scratch_types=(
        [pltpu.VMEM((SCH,), jnp.int32)] * 4
        + [pltpu.VMEM((SCH, HID), jnp.float32)] * 4
        + [pltpu.SemaphoreType.DMA] * 6
    ),
)
def _sc_gather_sub(tab_hbm, g_hbm, src_hbm, rev_hbm, out_hbm,
                   a0, a1, r0, r1, t0, t1, g0, g1,
                   si0, si1, sg0, sg1, so0, so1):
    c = lax.axis_index("c")
    s = lax.axis_index("s")
    base0 = (c * NS + s) * EPW
    sidx_v = (a0, a1)
    ridx_v = (r0, r1)
    t_v = (t0, t1)
    g_v = (g0, g1)
    sem_i = (si0, si1)
    sem_g = (sg0, sg1)
    sem_o = (so0, so1)

    def cbase(k):
        return pl.multiple_of(base0 + k * SCH, 8)

    pltpu.async_copy(src_hbm.at[pl.ds(cbase(0), SCH)], a0, si0)
    pltpu.async_copy(rev_hbm.at[pl.ds(cbase(0), SCH)], r0, si0)

    def pair(g, carry):
        for b in (0, 1):
            k = 2 * g + b
            nb = 1 - b

            @pl.when(k < SNC)
            def _():
                pltpu.make_async_copy(
                    src_hbm.at[pl.ds(cbase(k), SCH)], sidx_v[b], sem_i[b]).wait()
                pltpu.make_async_copy(
                    rev_hbm.at[pl.ds(cbase(k), SCH)], ridx_v[b], sem_i[b]).wait()

                @pl.when(k >= 2)
                def _():
                    pltpu.make_async_copy(
                        t_v[b], out_hbm.at[pl.ds(cbase(k - 2), SCH)],
                        sem_o[b]).wait()

                cps = []
                for off, ln in _SUBG:
                    cps.append(pltpu.async_copy(
                        tab_hbm.at[sidx_v[b].at[pl.ds(off, ln)]],
                        t_v[b].at[pl.ds(off, ln)], sem_g[b]))
                    cps.append(pltpu.async_copy(
                        g_hbm.at[ridx_v[b].at[pl.ds(off, ln)]],
                        g_v[b].at[pl.ds(off, ln)], sem_g[b]))

                @pl.when(k + 1 < SNC)
                def _():
                    pltpu.async_copy(
                        src_hbm.at[pl.ds(cbase(k + 1), SCH)], sidx_v[nb], sem_i[nb])
                    pltpu.async_copy(
                        rev_hbm.at[pl.ds(cbase(k + 1), SCH)], ridx_v[nb], sem_i[nb])

                @pl.when(k >= 1)
                def _():
                    def row2(i, carry2):
                        for r in range(2):
                            for j in range(HID // 16):
                                sl = pl.ds(j * 16, 16)
                                t_v[nb][i * 2 + r, sl] = (
                                    t_v[nb][i * 2 + r, sl]
                                    - g_v[nb][i * 2 + r, sl])
                        return carry2

                    lax.fori_loop(0, SCH // 2, row2, 0)
                    pltpu.async_copy(
                        t_v[nb], out_hbm.at[pl.ds(cbase(k - 1), SCH)], sem_o[nb])

                for cp in cps:
                    cp.wait()

            @pl.when(k == SNC)
            def _():
                def row2(i, carry2):
                    for r in range(2):
                        for j in range(HID // 16):
                            sl = pl.ds(j * 16, 16)
                            t_v[nb][i * 2 + r, sl] = (
                                t_v[nb][i * 2 + r, sl] - g_v[nb][i * 2 + r, sl])
                    return carry2

                lax.fori_loop(0, SCH // 2, row2, 0)
                pltpu.async_copy(
                    t_v[nb], out_hbm.at[pl.ds(cbase(SNC - 1), SCH)], sem_o[nb])
        return carry

    lax.fori_loop(0, (SNC + 2) // 2, pair, 0)
    pltpu.make_async_copy(
        t_v[1], out_hbm.at[pl.ds(cbase(SNC - 2), SCH)], sem_o[1]).wait()
    pltpu.make_async_copy(
        t_v[0], out_hbm.at[pl.ds(cbase(SNC - 1), SCH)], sem_o[0]).wait()


def kernel(x, edge_index, edge_attr, rev_edge_index, W_i, b_i, W_h, b_h, W_o, b_o):
    src = edge_index[0]
    dst = edge_index[1]
    wxt = jnp.transpose(W_i[:, :D_NODE_DIM])
    wet = jnp.transpose(W_i[:, D_NODE_DIM:])
    wht = jnp.transpose(W_h)
    wo1t = jnp.transpose(W_o[:, :D_NODE_DIM])
    wo2t = jnp.transpose(W_o[:, D_NODE_DIM:])
    bi2 = b_i.reshape(1, HID)
    bh2 = b_h.reshape(1, HID)
    bo2 = b_o.reshape(1, HID)

    p = _tc_matmul(x, wxt, block=2000)
    q = _tc_matmul(edge_attr, wet, bias=bi2)
    d = _sc_gather(p, src)
    g = _mm_relu_add(q, d, wht)
    for t in range(2):
        ab = _sc_scatter(g, dst)
        tt = _combine(ab, p, bh2)
        d = _sc_gather_sub(tt, g, src, rev_edge_index)
        if t == 0:
            g = _mm_relu_add(q, d, wht)
    ab = _sc_scatter_relu(q, d, dst)
    return _final(x, ab, wo1t, wo2t, bo2)


# speedup vs baseline: 1.0186x; 1.0009x over previous
import functools

import jax
import jax.numpy as jnp
from jax import lax
from jax.experimental import pallas as pl
from jax.experimental.pallas import tpu as pltpu
from jax.experimental.pallas import tpu_sc as plsc

N = 10000
E = 320000
D_NODE_DIM = 128
HID = 128
NC = 2
NS = 16
NW = NC * NS
EPW = E // NW
CHUNK = 80
NCHUNK = EPW // CHUNK
GCH = 400
GNC = EPW // GCH
SCH = 200
SNC = EPW // SCH
ROWS_A = 632
ROWS_B = N - (NS - 1) * ROWS_A

_sc_mesh = plsc.VectorSubcoreMesh(core_axis_name="c", subcore_axis_name="s")


def _mm_bias_body(a_ref, w_ref, b_ref, o_ref):
    o_ref[...] = (
        jnp.dot(a_ref[...], w_ref[...], preferred_element_type=jnp.float32)
        + b_ref[...]
    )


def _mm_body(a_ref, w_ref, o_ref):
    o_ref[...] = jnp.dot(a_ref[...], w_ref[...], preferred_element_type=jnp.float32)


def _tc_matmul(a, w, bias=None, block=16000):
    m, k = a.shape
    n = w.shape[1]
    grid = (m // block,)
    in_specs = [
        pl.BlockSpec((block, k), lambda i: (i, 0)),
        pl.BlockSpec((k, n), lambda i: (0, 0)),
    ]
    args = [a, w]
    body = _mm_body
    if bias is not None:
        in_specs.append(pl.BlockSpec((1, n), lambda i: (0, 0)))
        args.append(bias)
        body = _mm_bias_body
    return pl.pallas_call(
        body,
        grid=grid,
        in_specs=in_specs,
        out_specs=pl.BlockSpec((block, n), lambda i: (i, 0)),
        out_shape=jax.ShapeDtypeStruct((m, n), jnp.float32),
    )(*args)


def _mm_relu_add_body(q_ref, d_ref, w_ref, o_ref):
    h = jnp.maximum(q_ref[...] + d_ref[...], 0.0)
    o_ref[...] = jnp.dot(h, w_ref[...], preferred_element_type=jnp.float32)


def _mm_relu_add(q, d, w, block=16000):
    m = q.shape[0]
    n = w.shape[1]
    grid = (m // block,)
    return pl.pallas_call(
        _mm_relu_add_body,
        grid=grid,
        in_specs=[
            pl.BlockSpec((block, HID), lambda i: (i, 0)),
            pl.BlockSpec((block, HID), lambda i: (i, 0)),
            pl.BlockSpec((HID, n), lambda i: (0, 0)),
        ],
        out_specs=pl.BlockSpec((block, n), lambda i: (i, 0)),
        out_shape=jax.ShapeDtypeStruct((m, n), jnp.float32),
    )(q, d, w)


def _combine_body(ab_ref, p_ref, bh_ref, t_ref):
    t_ref[...] = ab_ref[0] + ab_ref[1] + p_ref[...] + bh_ref[...]


def _combine(ab, p, bh, block=5000):
    grid = (N // block,)
    return pl.pallas_call(
        _combine_body,
        grid=grid,
        in_specs=[
            pl.BlockSpec((NC, block, HID), lambda i: (0, i, 0)),
            pl.BlockSpec((block, HID), lambda i: (i, 0)),
            pl.BlockSpec((1, HID), lambda i: (0, 0)),
        ],
        out_specs=pl.BlockSpec((block, HID), lambda i: (i, 0)),
        out_shape=jax.ShapeDtypeStruct((N, HID), jnp.float32),
    )(ab, p, bh)


def _final_body(x_ref, ab_ref, w1_ref, w2_ref, b_ref, o_ref):
    f = ab_ref[0] + ab_ref[1]
    cond = jnp.sum(f, axis=1, keepdims=True) == 0.0
    mp = jnp.where(cond, x_ref[...], f)
    o_ref[...] = jax.nn.relu(
        jnp.dot(x_ref[...], w1_ref[...], preferred_element_type=jnp.float32)
        + jnp.dot(mp, w2_ref[...], preferred_element_type=jnp.float32)
        + b_ref[...]
    )


def _final(x, ab, w1t, w2t, bo, block=5000):
    grid = (N // block,)
    return pl.pallas_call(
        _final_body,
        grid=grid,
        in_specs=[
            pl.BlockSpec((block, D_NODE_DIM), lambda i: (i, 0)),
            pl.BlockSpec((NC, block, HID), lambda i: (0, i, 0)),
            pl.BlockSpec((D_NODE_DIM, HID), lambda i: (0, 0)),
            pl.BlockSpec((HID, HID), lambda i: (0, 0)),
            pl.BlockSpec((1, HID), lambda i: (0, 0)),
        ],
        out_specs=pl.BlockSpec((block, HID), lambda i: (i, 0)),
        out_shape=jax.ShapeDtypeStruct((N, HID), jnp.float32),
    )(x, ab, w1t, w2t, bo)


def _zero_acc(zbuf, acc_sh, s, row_off, zsem):
    def zrow(i, carry):
        for j in range(HID // 16):
            zbuf[i, pl.ds(j * 16, 16)] = jnp.zeros((16,), jnp.float32)
        return carry

    lax.fori_loop(0, CHUNK, zrow, 0)

    @pl.when(s < NS - 1)
    def _():
        def zfill(i, carry):
            off = pl.multiple_of(s * ROWS_A + i * CHUNK, 8)
            pltpu.async_copy(zbuf, acc_sh.at[pl.ds(off, CHUNK)], zsem)
            return carry
        lax.fori_loop(0, ROWS_A // CHUNK, zfill, 0)
        pltpu.async_copy(zbuf.at[pl.ds(0, ROWS_A % CHUNK)],
                         acc_sh.at[pl.ds(pl.multiple_of(
                             s * ROWS_A + (ROWS_A // CHUNK) * CHUNK, 8),
                             ROWS_A % CHUNK)], zsem)
        def zdrain(i, carry):
            off = pl.multiple_of(s * ROWS_A + i * CHUNK, 8)
            pltpu.make_async_copy(
                zbuf, acc_sh.at[pl.ds(off, CHUNK)], zsem).wait()
            return carry
        lax.fori_loop(0, ROWS_A // CHUNK, zdrain, 0)
        pltpu.make_async_copy(
            zbuf.at[pl.ds(0, ROWS_A % CHUNK)],
            acc_sh.at[pl.ds(pl.multiple_of(
                s * ROWS_A + (ROWS_A // CHUNK) * CHUNK, 8),
                ROWS_A % CHUNK)], zsem).wait()

    @pl.when(s == NS - 1)
    def _():
        base_b = (NS - 1) * ROWS_A

        def zfill(i, carry):
            off = pl.multiple_of(base_b + i * CHUNK, 8)
            pltpu.async_copy(zbuf, acc_sh.at[pl.ds(off, CHUNK)], zsem)
            return carry
        lax.fori_loop(0, ROWS_B // CHUNK, zfill, 0)
        pltpu.async_copy(zbuf.at[pl.ds(0, ROWS_B % CHUNK)],
                         acc_sh.at[pl.ds(base_b + (ROWS_B // CHUNK) * CHUNK,
                                         ROWS_B % CHUNK)], zsem)
        def zdrain(i, carry):
            off = pl.multiple_of(base_b + i * CHUNK, 8)
            pltpu.make_async_copy(
                zbuf, acc_sh.at[pl.ds(off, CHUNK)], zsem).wait()
            return carry
        lax.fori_loop(0, ROWS_B // CHUNK, zdrain, 0)
        pltpu.make_async_copy(
            zbuf.at[pl.ds(0, ROWS_B % CHUNK)],
            acc_sh.at[pl.ds(base_b + (ROWS_B // CHUNK) * CHUNK,
                            ROWS_B % CHUNK)], zsem).wait()


def _readout_acc(acc_sh, out_hbm, c, s, row_off):
    @pl.when(s < NS - 1)
    def _():
        pltpu.sync_copy(acc_sh.at[pl.ds(row_off, ROWS_A)],
                        out_hbm.at[c, pl.ds(row_off, ROWS_A)])

    @pl.when(s == NS - 1)
    def _():
        pltpu.sync_copy(acc_sh.at[pl.ds((NS - 1) * ROWS_A, ROWS_B)],
                        out_hbm.at[c, pl.ds((NS - 1) * ROWS_A, ROWS_B)])


@functools.partial(
    pl.kernel,
    out_type=jax.ShapeDtypeStruct((NC, N, HID), jnp.float32),
    mesh=_sc_mesh,
    scratch_types=[
        pltpu.VMEM((CHUNK,), jnp.int32),
        pltpu.VMEM((CHUNK,), jnp.int32),
        pltpu.VMEM((CHUNK,), jnp.int32),
        pltpu.VMEM((CHUNK, HID), jnp.float32),
        pltpu.VMEM((CHUNK, HID), jnp.float32),
        pltpu.VMEM((CHUNK, HID), jnp.float32),
        pltpu.VMEM_SHARED((N, HID), jnp.float32),
        pltpu.SemaphoreType.DMA,
        pltpu.SemaphoreType.DMA,
        pltpu.SemaphoreType.DMA,
        pltpu.SemaphoreType.DMA,
        pltpu.SemaphoreType.DMA,
        pltpu.SemaphoreType.DMA,
    ],
)
def _sc_scatter(rows_hbm, dst_hbm, out_hbm,
                idx0, idx1, idx2, rows0, rows1, rows2, acc_sh,
                sem_l0, sem_l1, sem_l2, sem_s0, sem_s1, sem_s2):
    c = lax.axis_index("c")
    s = lax.axis_index("s")
    wid = c * NS + s
    row_off = pl.multiple_of(s * ROWS_A, 8)
    idx_v = (idx0, idx1, idx2)
    rows_v = (rows0, rows1, rows2)
    sem_l = (sem_l0, sem_l1, sem_l2)
    sem_s = (sem_s0, sem_s1, sem_s2)

    _zero_acc(rows0, acc_sh, s, row_off, sem_s0)
    plsc.subcore_barrier()

    base0 = wid * EPW

    def cbase(k):
        return pl.multiple_of(base0 + k * CHUNK, 8)

    pltpu.async_copy(dst_hbm.at[pl.ds(cbase(0), CHUNK)], idx0, sem_l0)
    pltpu.async_copy(rows_hbm.at[pl.ds(cbase(0), CHUNK)], rows0, sem_l0)

    def trip(g, carry):
        for b in (0, 1, 2):
            k = 3 * g + b
            nxt = (b + 1) % 3

            @pl.when(k < NCHUNK)
            def _():
                pltpu.make_async_copy(
                    dst_hbm.at[pl.ds(cbase(k), CHUNK)], idx_v[b], sem_l[b]).wait()
                pltpu.make_async_copy(
                    rows_hbm.at[pl.ds(cbase(k), CHUNK)], rows_v[b], sem_l[b]).wait()

                @pl.when(k >= 2)
                def _():
                    pltpu.make_async_copy(
                        rows_hbm.at[pl.ds(cbase(0), CHUNK)], rows_v[nxt],
                        sem_s[nxt]).wait()

                @pl.when(k + 1 < NCHUNK)
                def _():
                    pltpu.async_copy(
                        dst_hbm.at[pl.ds(cbase(k + 1), CHUNK)], idx_v[nxt],
                        sem_l[nxt])
                    pltpu.async_copy(
                        rows_hbm.at[pl.ds(cbase(k + 1), CHUNK)], rows_v[nxt],
                        sem_l[nxt])

                pltpu.async_copy(
                    rows_v[b], acc_sh.at[idx_v[b]], sem_s[b], add=True)
        return carry

    lax.fori_loop(0, (NCHUNK + 2) // 3, trip, 0)
    pltpu.make_async_copy(
        rows_hbm.at[pl.ds(cbase(0), CHUNK)],
        rows_v[(NCHUNK - 2) % 3], sem_s[(NCHUNK - 2) % 3]).wait()
    pltpu.make_async_copy(
        rows_hbm.at[pl.ds(cbase(0), CHUNK)],
        rows_v[(NCHUNK - 1) % 3], sem_s[(NCHUNK - 1) % 3]).wait()

    plsc.subcore_barrier()
    _readout_acc(acc_sh, out_hbm, c, s, row_off)


@functools.partial(
    pl.kernel,
    out_type=jax.ShapeDtypeStruct((NC, N, HID), jnp.float32),
    mesh=_sc_mesh,
    scratch_types=[
        pltpu.VMEM((CHUNK,), jnp.int32),
        pltpu.VMEM((CHUNK,), jnp.int32),
        pltpu.VMEM((CHUNK, HID), jnp.float32),
        pltpu.VMEM((CHUNK, HID), jnp.float32),
        pltpu.VMEM((CHUNK, HID), jnp.float32),
        pltpu.VMEM((CHUNK, HID), jnp.float32),
        pltpu.VMEM_SHARED((N, HID), jnp.float32),
        pltpu.SemaphoreType.DMA,
        pltpu.SemaphoreType.DMA,
        pltpu.SemaphoreType.DMA,
        pltpu.SemaphoreType.DMA,
    ],
)
def _sc_scatter_relu(q_hbm, d_hbm, dst_hbm, out_hbm,
                     idx0, idx1, q0, q1, d0, d1, acc_sh,
                     sem_l0, sem_l1, sem_s0, sem_s1):
    c = lax.axis_index("c")
    s = lax.axis_index("s")
    wid = c * NS + s
    row_off = pl.multiple_of(s * ROWS_A, 8)
    idx_v = (idx0, idx1)
    q_v = (q0, q1)
    d_v = (d0, d1)
    sem_l = (sem_l0, sem_l1)
    sem_s = (sem_s0, sem_s1)

    _zero_acc(q0, acc_sh, s, row_off, sem_s0)
    plsc.subcore_barrier()

    base0 = wid * EPW

    def cbase(k):
        return pl.multiple_of(base0 + k * CHUNK, 8)

    pltpu.async_copy(dst_hbm.at[pl.ds(cbase(0), CHUNK)], idx0, sem_l0)
    pltpu.async_copy(q_hbm.at[pl.ds(cbase(0), CHUNK)], q0, sem_l0)
    pltpu.async_copy(d_hbm.at[pl.ds(cbase(0), CHUNK)], d0, sem_l0)

    def pair(g, carry):
        for b in (0, 1):
            k = 2 * g + b
            nb = 1 - b

            @pl.when(k < NCHUNK)
            def _():
                pltpu.make_async_copy(
                    dst_hbm.at[pl.ds(cbase(k), CHUNK)], idx_v[b], sem_l[b]).wait()
                pltpu.make_async_copy(
                    q_hbm.at[pl.ds(cbase(k), CHUNK)], q_v[b], sem_l[b]).wait()
                pltpu.make_async_copy(
                    d_hbm.at[pl.ds(cbase(k), CHUNK)], d_v[b], sem_l[b]).wait()

                @pl.when(k >= 1)
                def _():
                    pltpu.make_async_copy(
                        q_hbm.at[pl.ds(cbase(0), CHUNK)], q_v[nb],
                        sem_s[nb]).wait()

                @pl.when(k + 1 < NCHUNK)
                def _():
                    pltpu.async_copy(
                        dst_hbm.at[pl.ds(cbase(k + 1), CHUNK)], idx_v[nb], sem_l[nb])
                    pltpu.async_copy(
                        q_hbm.at[pl.ds(cbase(k + 1), CHUNK)], q_v[nb], sem_l[nb])
                    pltpu.async_copy(
                        d_hbm.at[pl.ds(cbase(k + 1), CHUNK)], d_v[nb], sem_l[nb])

                def row2(i, carry2):
                    for r in range(2):
                        for j in range(HID // 16):
                            sl = pl.ds(j * 16, 16)
                            q_v[b][i * 2 + r, sl] = jnp.maximum(
                                q_v[b][i * 2 + r, sl] + d_v[b][i * 2 + r, sl], 0.0)
                    return carry2

                lax.fori_loop(0, CHUNK // 2, row2, 0)

                pltpu.async_copy(
                    q_v[b], acc_sh.at[idx_v[b]], sem_s[b], add=True)
        return carry

    lax.fori_loop(0, (NCHUNK + 1) // 2, pair, 0)
    pltpu.make_async_copy(
        q_hbm.at[pl.ds(cbase(0), CHUNK)], q_v[0], sem_s[0]).wait()

    plsc.subcore_barrier()
    _readout_acc(acc_sh, out_hbm, c, s, row_off)


@functools.partial(
    pl.kernel,
    out_type=jax.ShapeDtypeStruct((E, HID), jnp.float32),
    mesh=_sc_mesh,
    scratch_types=(
        [pltpu.VMEM((GCH,), jnp.int32)] * 2
        + [pltpu.VMEM((GCH, HID), jnp.float32)] * 2
        + [pltpu.SemaphoreType.DMA] * 6
    ),
)
def _sc_gather(tab_hbm, src_hbm, out_hbm,
               i0, i1, t0, t1,
               si0, si1, sg0, sg1, so0, so1):
    c = lax.axis_index("c")
    s = lax.axis_index("s")
    base0 = (c * NS + s) * EPW
    idx_v = (i0, i1)
    t_v = (t0, t1)
    sem_i = (si0, si1)
    sem_g = (sg0, sg1)
    sem_o = (so0, so1)

    def cbase(k):
        return pl.multiple_of(base0 + k * GCH, 8)

    pltpu.async_copy(src_hbm.at[pl.ds(cbase(0), GCH)], i0, si0)

    def pair(g, carry):
        for b in (0, 1):
            k = 2 * g + b
            nb = 1 - b

            @pl.when(k < GNC)
            def _():
                pltpu.make_async_copy(
                    src_hbm.at[pl.ds(cbase(k), GCH)], idx_v[b], sem_i[b]).wait()

                @pl.when(k >= 2)
                def _():
                    pltpu.make_async_copy(
                        t_v[b], out_hbm.at[pl.ds(cbase(k - 2), GCH)],
                        sem_o[b]).wait()

                cps = [
                    pltpu.async_copy(
                        tab_hbm.at[idx_v[b].at[pl.ds(j * 80, 80)]],
                        t_v[b].at[pl.ds(j * 80, 80)], sem_g[b])
                    for j in range(GCH // 80)
                ]

                @pl.when(k + 1 < GNC)
                def _():
                    pltpu.async_copy(
                        src_hbm.at[pl.ds(cbase(k + 1), GCH)], idx_v[nb], sem_i[nb])

                @pl.when(k >= 1)
                def _():
                    pltpu.async_copy(
                        t_v[nb], out_hbm.at[pl.ds(cbase(k - 1), GCH)], sem_o[nb])

                for cp in cps:
                    cp.wait()

            @pl.when(k == GNC)
            def _():
                pltpu.async_copy(
                    t_v[nb], out_hbm.at[pl.ds(cbase(GNC - 1), GCH)], sem_o[nb])
        return carry

    lax.fori_loop(0, (GNC + 2) // 2, pair, 0)
    pltpu.make_async_copy(
        t_v[1], out_hbm.at[pl.ds(cbase(GNC - 2), GCH)], sem_o[1]).wait()
    pltpu.make_async_copy(
        t_v[0], out_hbm.at[pl.ds(cbase(GNC - 1), GCH)], sem_o[0]).wait()


_SUBG = ((0, 80), (80, 80), (160, 40))


@functools.partial(
    pl.kernel,
    out_type=jax.ShapeDtypeStruct((E, HID), jnp.float32),
    mesh=_sc_mesh,
    scratch_types=(
        [pltpu.VMEM((SCH,), jnp.int32)] * 4
        + [pltpu.VMEM((SCH, HID), jnp.float32)] * 4
        + [pltpu.SemaphoreType.DMA] * 6
    ),
)
def _sc_gather_sub(tab_hbm, g_hbm, src_hbm, rev_hbm, out_hbm,
                   a0, a1, r0, r1, t0, t1, g0, g1,
                   si0, si1, sg0, sg1, so0, so1):
    c = lax.axis_index("c")
    s = lax.axis_index("s")
    base0 = (c * NS + s) * EPW
    sidx_v = (a0, a1)
    ridx_v = (r0, r1)
    t_v = (t0, t1)
    g_v = (g0, g1)
    sem_i = (si0, si1)
    sem_g = (sg0, sg1)
    sem_o = (so0, so1)

    def cbase(k):
        return pl.multiple_of(base0 + k * SCH, 8)

    pltpu.async_copy(src_hbm.at[pl.ds(cbase(0), SCH)], a0, si0)
    pltpu.async_copy(rev_hbm.at[pl.ds(cbase(0), SCH)], r0, si0)

    def pair(g, carry):
        for b in (0, 1):
            k = 2 * g + b
            nb = 1 - b

            @pl.when(k < SNC)
            def _():
                pltpu.make_async_copy(
                    src_hbm.at[pl.ds(cbase(k), SCH)], sidx_v[b], sem_i[b]).wait()
                pltpu.make_async_copy(
                    rev_hbm.at[pl.ds(cbase(k), SCH)], ridx_v[b], sem_i[b]).wait()

                @pl.when(k >= 2)
                def _():
                    pltpu.make_async_copy(
                        t_v[b], out_hbm.at[pl.ds(cbase(k - 2), SCH)],
                        sem_o[b]).wait()

                cps = []
                for off, ln in _SUBG:
                    cps.append(pltpu.async_copy(
                        tab_hbm.at[sidx_v[b].at[pl.ds(off, ln)]],
                        t_v[b].at[pl.ds(off, ln)], sem_g[b]))
                    cps.append(pltpu.async_copy(
                        g_hbm.at[ridx_v[b].at[pl.ds(off, ln)]],
                        g_v[b].at[pl.ds(off, ln)], sem_g[b]))

                @pl.when(k + 1 < SNC)
                def _():
                    pltpu.async_copy(
                        src_hbm.at[pl.ds(cbase(k + 1), SCH)], sidx_v[nb], sem_i[nb])
                    pltpu.async_copy(
                        rev_hbm.at[pl.ds(cbase(k + 1), SCH)], ridx_v[nb], sem_i[nb])

                @pl.when(k >= 1)
                def _():
                    def row2(i, carry2):
                        for r in range(2):
                            for j in range(HID // 16):
                                sl = pl.ds(j * 16, 16)
                                t_v[nb][i * 2 + r, sl] = (
                                    t_v[nb][i * 2 + r, sl]
                                    - g_v[nb][i * 2 + r, sl])
                        return carry2

                    lax.fori_loop(0, SCH // 2, row2, 0)
                    pltpu.async_copy(
                        t_v[nb], out_hbm.at[pl.ds(cbase(k - 1), SCH)], sem_o[nb])

                for cp in cps:
                    cp.wait()

            @pl.when(k == SNC)
            def _():
                def row2(i, carry2):
                    for r in range(2):
                        for j in range(HID // 16):
                            sl = pl.ds(j * 16, 16)
                            t_v[nb][i * 2 + r, sl] = (
                                t_v[nb][i * 2 + r, sl] - g_v[nb][i * 2 + r, sl])
                    return carry2

                lax.fori_loop(0, SCH // 2, row2, 0)
                pltpu.async_copy(
                    t_v[nb], out_hbm.at[pl.ds(cbase(SNC - 1), SCH)], sem_o[nb])
        return carry

    lax.fori_loop(0, (SNC + 2) // 2, pair, 0)
    pltpu.make_async_copy(
        t_v[1], out_hbm.at[pl.ds(cbase(SNC - 2), SCH)], sem_o[1]).wait()
    pltpu.make_async_copy(
        t_v[0], out_hbm.at[pl.ds(cbase(SNC - 1), SCH)], sem_o[0]).wait()


def kernel(x, edge_index, edge_attr, rev_edge_index, W_i, b_i, W_h, b_h, W_o, b_o):
    src = edge_index[0]
    dst = edge_index[1]
    wxt = jnp.transpose(W_i[:, :D_NODE_DIM])
    wet = jnp.transpose(W_i[:, D_NODE_DIM:])
    wht = jnp.transpose(W_h)
    wo1t = jnp.transpose(W_o[:, :D_NODE_DIM])
    wo2t = jnp.transpose(W_o[:, D_NODE_DIM:])
    bi2 = b_i.reshape(1, HID)
    bh2 = b_h.reshape(1, HID)
    bo2 = b_o.reshape(1, HID)

    p = _tc_matmul(x, wxt, block=2000)
    q = _tc_matmul(edge_attr, wet, bias=bi2)
    d = _sc_gather(p, src)
    g = _mm_relu_add(q, d, wht)
    for t in range(2):
        ab = _sc_scatter(g, dst)
        tt = _combine(ab, p, bh2)
        d = _sc_gather_sub(tt, g, src, rev_edge_index)
        if t == 0:
            g = _mm_relu_add(q, d, wht)
    ab = _sc_scatter_relu(q, d, dst)
    return _final(x, ab, wo1t, wo2t, bo2)
